# baseline (speedup 1.0000x reference)
import functools

import jax
import jax.numpy as jnp
from jax import lax
from jax.experimental import pallas as pl
from jax.experimental.pallas import tpu as pltpu
from jax.experimental.pallas import tpu_sc as plsc

N_NODES = 10000
N_EDGES = 160000
H = 512

NC, NS = 2, 16
NW = NC * NS

_mesh = functools.partial(
    plsc.VectorSubcoreMesh, core_axis_name="c", subcore_axis_name="s",
    num_cores=NC, num_subcores=NS)


def _gelu(x):
    return x * 0.5 * (1.0 + lax.erf(x * 0.7071067811865476))


def _ln(x, g, b):
    m = jnp.mean(x, axis=-1, keepdims=True)
    v = jnp.mean((x - m) ** 2, axis=-1, keepdims=True)
    return (x - m) * lax.rsqrt(v + 1e-5) * g + b


def _mmT(a, w):
    return lax.dot_general(a, w, (((1,), (1,)), ((), ())),
                           preferred_element_type=jnp.float32)


def _mmT16(a, w):
    return lax.dot_general(a.astype(jnp.bfloat16), w.astype(jnp.bfloat16),
                           (((1,), (1,)), ((), ())),
                           preferred_element_type=jnp.float32)


def _sc_gather(table, idx):
    V, D = table.shape
    E = idx.shape[0]
    per_w = E // NW
    C = 64
    NB = 3
    n_full = per_w // C
    n_tri = n_full // NB
    rem = per_w - n_full * C
    assert n_tri * NB == n_full

    def body(table_hbm, idx_hbm, out_hbm, idx_all, r0, r1, r2,
             g0, g1, g2, w0, w1, w2):
        wid = lax.axis_index("s") * NC + lax.axis_index("c")
        base = wid * per_w
        rows = (r0, r1, r2)
        gsem = (g0, g1, g2)
        wsem = (w0, w1, w2)

        pltpu.sync_copy(idx_hbm.at[pl.ds(base, per_w)], idx_all)

        def gd(j, b):
            return pltpu.make_async_copy(
                table_hbm.at[idx_all.at[pl.ds(j * C, C)]], rows[b], gsem[b])

        def wd(j, b):
            return pltpu.make_async_copy(
                rows[b], out_hbm.at[pl.ds(base + j * C, C)], wsem[b])

        for b in range(NB):
            gd(b, b).start()

        def tri(i, carry):
            for b in range(NB):
                j = i * NB + b
                gd(j, b).wait()
                wd(j, b).start()
                jn = j + NB

                @pl.when(jn < n_full)
                def _():
                    wd(j, b).wait()
                    gd(jn, b).start()

            return carry

        lax.fori_loop(0, n_tri, tri, 0)
        for b in range(NB):
            wd(n_full - NB + b, b).wait()
        if rem:
            off = base + n_full * C
            pltpu.async_copy(table_hbm.at[idx_all.at[pl.ds(n_full * C, rem)]],
                             rows[0].at[pl.ds(0, rem)], gsem[0]).wait()
            pltpu.sync_copy(rows[0].at[pl.ds(0, rem)],
                            out_hbm.at[pl.ds(off, rem)])

    return pl.kernel(
        body,
        out_type=jax.ShapeDtypeStruct((E, D), jnp.float32),
        mesh=_mesh(),
        scratch_types=[
            pltpu.VMEM((per_w,), jnp.int32),
        ] + [pltpu.VMEM((C, D), jnp.float32)] * 3
        + [pltpu.SemaphoreType.DMA] * 6,
    )(table, idx)


def _sc_scatter_add(m2_chunks, col, n_nodes):
    E = col.shape[0]
    F = 128
    per_t = E // NS
    C = 128
    n_full = per_t // C
    rem = per_t - n_full * C
    SL = 64
    vp = ((n_nodes + NS * SL - 1) // (NS * SL)) * (NS * SL)
    rows_t = vp // NS
    n_sl = rows_t // SL

    def body(m_all, col_hbm, out_hbm,
             accum, zbuf, mb0, mb1, ib0, ib1, irem, mrem,
             is0, is1, ms0, ms1, as0, as1):
        c = lax.axis_index("c")
        s = lax.axis_index("s")
        mbuf = (mb0, mb1)
        ibuf = (ib0, ib1)
        isem = (is0, is1)
        msem = (ms0, ms1)
        asem = (as0, as1)

        zv = jnp.zeros((16,), jnp.float32)

        def zrow(i, carry):
            for j in range(F // 16):
                zbuf[i, pl.ds(j * 16, 16)] = zv
            return carry

        lax.fori_loop(0, SL, zrow, 0)

        for k in range(2):
            q = 2 * c + k

            def zslice(t, carry):
                pltpu.sync_copy(zbuf, accum.at[pl.ds(s * rows_t + t * SL, SL)])
                return carry

            lax.fori_loop(0, n_sl, zslice, 0)
            plsc.subcore_barrier()

            ebase = s * per_t

            def ld(j, b):
                off = ebase + j * C
                return (pltpu.make_async_copy(
                            col_hbm.at[pl.ds(off, C)], ibuf[b], isem[b]),
                        pltpu.make_async_copy(
                            m_all.at[q, pl.ds(off, C)], mbuf[b], msem[b]))

            def ad(b):
                return pltpu.make_async_copy(
                    mbuf[b], accum.at[ibuf[b]], asem[b])

            for b in range(2):
                di, dm = ld(b, b)
                di.start()
                dm.start()

            def step(i, carry):
                for b in range(2):
                    j = 2 * i + b
                    di, dm = ld(j, b)
                    di.wait()
                    dm.wait()
                    ad(b).start(add=True)
                    jn = j + 2

                    @pl.when(jn < n_full)
                    def _():
                        ad(b).wait()
                        di2, dm2 = ld(jn, b)
                        di2.start()
                        dm2.start()

                return carry

            lax.fori_loop(0, n_full // 2, step, 0)
            for b in range(2):
                ad(b).wait()
            if rem:
                off = ebase + n_full * C
                pltpu.sync_copy(col_hbm.at[pl.ds(off, rem)], irem)
                pltpu.sync_copy(m_all.at[q, pl.ds(off, rem)], mrem)
                pltpu.sync_copy(mrem, accum.at[irem], add=True)

            plsc.subcore_barrier()
            def wslice(t, carry):
                off = s * rows_t + t * SL
                pltpu.sync_copy(accum.at[pl.ds(off, SL)], zbuf)
                pltpu.sync_copy(zbuf, out_hbm.at[q, pl.ds(off, SL)])
                return carry

            lax.fori_loop(0, n_sl, wslice, 0)
            lax.fori_loop(0, SL, zrow, 0)
            plsc.subcore_barrier()

    return pl.kernel(
        body,
        out_type=jax.ShapeDtypeStruct((4, vp, F), jnp.float32),
        mesh=_mesh(),
        scratch_types=[
            pltpu.VMEM_SHARED((vp, F), jnp.float32),
            pltpu.VMEM((SL, F), jnp.float32),
            pltpu.VMEM((C, F), jnp.float32),
            pltpu.VMEM((C, F), jnp.float32),
            pltpu.VMEM((C,), jnp.int32),
            pltpu.VMEM((C,), jnp.int32),
            pltpu.VMEM((16,), jnp.int32),
            pltpu.VMEM((16, F), jnp.float32),
        ] + [pltpu.SemaphoreType.DMA] * 6,
    )(m2_chunks, col)


def _sc_counts(col, n_nodes):
    E = col.shape[0]
    F = 128
    per_t = E // (NC * NS)
    C = 128
    n_full = per_t // C
    rem = per_t - n_full * C
    SL = 128
    vp = ((n_nodes + NS * SL - 1) // (NS * SL)) * (NS * SL)
    rows_t = vp // NS
    n_sl = rows_t // SL

    def body(col_hbm, out_hbm, accum, zbuf, ones_v, ib0, ib1, irem,
             is0, is1, as0, as1):
        c = lax.axis_index("c")
        s = lax.axis_index("s")
        ibuf = (ib0, ib1)
        isem = (is0, is1)
        asem = (as0, as1)
        zv = jnp.zeros((16,), jnp.float32)
        ov = jnp.ones((16,), jnp.float32)

        def zrow(i, carry):
            for j in range(F // 16):
                zbuf[i, pl.ds(j * 16, 16)] = zv
            return carry

        lax.fori_loop(0, SL, zrow, 0)

        def orow(i, carry):
            for j in range(F // 16):
                ones_v[i, pl.ds(j * 16, 16)] = ov
            return carry

        lax.fori_loop(0, C, orow, 0)

        def zslice(t, carry):
            pltpu.sync_copy(zbuf, accum.at[pl.ds(s * rows_t + t * SL, SL)])
            return carry

        lax.fori_loop(0, n_sl, zslice, 0)
        plsc.subcore_barrier()

        ebase = (c * NS + s) * per_t

        def ld(j, b):
            return pltpu.make_async_copy(
                col_hbm.at[pl.ds(ebase + j * C, C)], ibuf[b], isem[b])

        def ad(b):
            return pltpu.make_async_copy(ones_v, accum.at[ibuf[b]], asem[b])

        for b in range(2):
            ld(b, b).start()

        def step(i, carry):
            for b in range(2):
                j = 2 * i + b
                ld(j, b).wait()
                ad(b).start(add=True)
                jn = j + 2

                @pl.when(jn < n_full)
                def _():
                    ad(b).wait()
                    ld(jn, b).start()

            return carry

        lax.fori_loop(0, n_full // 2, step, 0)
        if n_full % 2:
            ld(n_full - 1, 0).wait()
            ad(0).start(add=True)
        for b in range(2):
            ad(b).wait()
        if rem:
            off = ebase + n_full * C
            pltpu.sync_copy(col_hbm.at[pl.ds(off, rem)], irem)
            pltpu.sync_copy(ones_v.at[pl.ds(0, rem)], accum.at[irem],
                            add=True)
        plsc.subcore_barrier()

        def wslice(t, carry):
            off = s * rows_t + t * SL
            pltpu.sync_copy(accum.at[pl.ds(off, SL)], zbuf)
            pltpu.sync_copy(zbuf, out_hbm.at[c, pl.ds(off, SL)])
            return carry

        lax.fori_loop(0, n_sl, wslice, 0)

    return pl.kernel(
        body,
        out_type=jax.ShapeDtypeStruct((NC, vp, F), jnp.float32),
        mesh=_mesh(),
        scratch_types=[
            pltpu.VMEM_SHARED((vp, F), jnp.float32),
            pltpu.VMEM((SL, F), jnp.float32),
            pltpu.VMEM((C, F), jnp.float32),
            pltpu.VMEM((C,), jnp.int32),
            pltpu.VMEM((C,), jnp.int32),
            pltpu.VMEM((max(rem, 8),), jnp.int32),
        ] + [pltpu.SemaphoreType.DMA] * 4,
    )(col)


BN = 2000
BE = 2000


def _full(shape):
    return pl.BlockSpec(shape, lambda i: (0,) * len(shape))


def _rows(block):
    return pl.BlockSpec(block, lambda i: (i,) + (0,) * (len(block) - 1))


def _node_encoder(mol_x_pad, aW, ab, ag, abeta, npW, npb, npg, npbeta,
                  w1a0):
    n = mol_x_pad.shape[0]
    K = mol_x_pad.shape[1]

    def body(x_ref, aW_r, ab_r, ag_r, abeta_r, npW_r, npb_r, npg_r,
             npbeta_r, w1a0_r, x0_ref, xa0_ref):
        a = _mmT(x_ref[...], aW_r[...]) + ab_r[...]
        a = _gelu(_ln(a, ag_r[...], abeta_r[...]))
        x0 = _mmT(a, npW_r[...]) + npb_r[...]
        x0 = _gelu(_ln(x0, npg_r[...], npbeta_r[...]))
        x0_ref[...] = x0
        xa0_ref[...] = _mmT(x0, w1a0_r[...])

    return pl.pallas_call(
        body,
        grid=(n // BN,),
        in_specs=[
            _rows((BN, K)),
            _full((H // 2, K)), _full((1, H // 2)), _full((1, H // 2)),
            _full((1, H // 2)),
            _full((H, H // 2)), _full((1, H)), _full((1, H)), _full((1, H)),
            _full((H, H)),
        ],
        out_specs=[_rows((BN, H)), _rows((BN, H))],
        out_shape=[jax.ShapeDtypeStruct((n, H), jnp.float32),
                   jax.ShapeDtypeStruct((n, H), jnp.float32)],
    )(mol_x_pad, aW, ab, ag, abeta, npW, npb, npg, npbeta, w1a0)


def _edge_mlp0(attr, scal, g0, bW, bb, bg, bbeta, s4, sb_w, sb, sg, sbeta,
               epW, epb, epg, epbeta, w1b, b1, g1, beta1, w2, b2, g2,
               beta2):
    e = attr.shape[0]

    def body(attr_r, scal_r, g_r, bW_r, bb_r, bg_r, bbeta_r, s4_r, sbw_r,
             sb_r, sg_r, sbeta_r, epW_r, epb_r, epg_r, epbeta_r, w1b_r,
             b1_r, g1_r, beta1_r, w2_r, b2_r, g2_r, beta2_r,
             ea_ref, m_ref):
        bond = _mmT(attr_r[...], bW_r[...]) + bb_r[...]
        bond = _gelu(_ln(bond, bg_r[...], bbeta_r[...]))
        spat = (_mmT(scal_r[...], s4_r[...])
                + _mmT(bond, sbw_r[...]) + sb_r[...])
        spat = _gelu(_ln(spat, sg_r[...], sbeta_r[...]))
        ea = _mmT16(spat, epW_r[...]) + epb_r[...]
        ea = _gelu(_ln(ea, epg_r[...], epbeta_r[...]))
        ea_ref[...] = ea
        t = g_r[...] + _mmT16(ea, w1b_r[...]) + b1_r[...]
        t = _gelu(_ln(t, g1_r[...], beta1_r[...]))
        m = _mmT16(t, w2_r[...]) + b2_r[...]
        m = _gelu(_ln(m, g2_r[...], beta2_r[...]))
        for q in range(4):
            m_ref[q] = m[:, q * 128:(q + 1) * 128]

    return pl.pallas_call(
        body,
        grid=(e // BE,),
        in_specs=[
            _rows((BE, 8)), _rows((BE, 8)), _rows((BE, H)),
            _full((128, 8)), _full((1, 128)), _full((1, 128)),
            _full((1, 128)),
            _full((256, 8)), _full((256, 128)), _full((1, 256)),
            _full((1, 256)), _full((1, 256)),
            _full((H, 256)), _full((1, H)), _full((1, H)), _full((1, H)),
            _full((H, H)), _full((1, H)), _full((1, H)), _full((1, H)),
            _full((H, H)), _full((1, H)), _full((1, H)), _full((1, H)),
        ],
        out_specs=[_rows((BE, H)),
                   pl.BlockSpec((4, BE, 128), lambda i: (0, i, 0))],
        out_shape=[jax.ShapeDtypeStruct((e, H), jnp.float32),
                   jax.ShapeDtypeStruct((4, e, 128), jnp.float32)],
    )(attr, scal, g0, bW, bb, bg, bbeta, s4, sb_w, sb, sg, sbeta, epW,
      epb, epg, epbeta, w1b, b1, g1, beta1, w2, b2, g2, beta2)


def _edge_mlp(ea, g, w1b, b1, g1, beta1, w2, b2, g2, beta2):
    e = ea.shape[0]

    def body(ea_r, g_r, w1b_r, b1_r, g1_r, beta1_r, w2_r, b2_r, g2_r,
             beta2_r, m_ref):
        t = g_r[...] + _mmT16(ea_r[...], w1b_r[...]) + b1_r[...]
        t = _gelu(_ln(t, g1_r[...], beta1_r[...]))
        m = _mmT16(t, w2_r[...]) + b2_r[...]
        m = _gelu(_ln(m, g2_r[...], beta2_r[...]))
        for q in range(4):
            m_ref[q] = m[:, q * 128:(q + 1) * 128]

    return pl.pallas_call(
        body,
        grid=(e // BE,),
        in_specs=[
            _rows((BE, H)), _rows((BE, H)),
            _full((H, H)), _full((1, H)), _full((1, H)), _full((1, H)),
            _full((H, H)), _full((1, H)), _full((1, H)), _full((1, H)),
        ],
        out_specs=pl.BlockSpec((4, BE, 128), lambda i: (0, i, 0)),
        out_shape=jax.ShapeDtypeStruct((4, e, 128), jnp.float32),
    )(ea, g, w1b, b1, g1, beta1, w2, b2, g2, beta2)


def _x_update(x, s_all, cnt, w1a):
    n = x.shape[0]

    def body(x_r, s_r, c_r, w1a_r, xn_ref, xa_ref):
        sv = s_r[...]
        s = jnp.concatenate([sv[q] for q in range(4)], axis=-1)
        cv = c_r[...]
        cnt = cv[0][:, 0:1] + cv[1][:, 0:1]
        xn = x_r[...] + s / jnp.maximum(cnt, 1.0)
        xn_ref[...] = xn
        xa_ref[...] = _mmT(xn, w1a_r[...])

    return pl.pallas_call(
        body,
        grid=(n // BN,),
        in_specs=[_rows((BN, H)),
                  pl.BlockSpec((4, BN, 128), lambda i: (0, i, 0)),
                  pl.BlockSpec((2, BN, 128), lambda i: (0, i, 0)),
                  _full((H, H))],
        out_specs=[_rows((BN, H)), _rows((BN, H))],
        out_shape=[jax.ShapeDtypeStruct((n, H), jnp.float32),
                   jax.ShapeDtypeStruct((n, H), jnp.float32)],
    )(x, s_all, cnt, w1a)


def _seq_branch(emb, seqW, seqb, wv0, bv0, wo0, bo0, g10, beta10, w10,
                b10, w20, b20, g20, beta20, wv1, bv1, wo1, bo1, g11,
                beta11, w11, b11, w21, b21, g21, beta21, fusWs, fusb):
    def body(emb_r, seqW_r, seqb_r, wv0_r, bv0_r, wo0_r, bo0_r, g10_r,
             beta10_r, w10_r, b10_r, w20_r, b20_r, g20_r, beta20_r,
             wv1_r, bv1_r, wo1_r, bo1_r, g11_r, beta11_r, w11_r, b11_r,
             w21_r, b21_r, g21_r, beta21_r, fusWs_r, fusb_r, c_ref):
        h = _mmT(emb_r[...], seqW_r[...]) + seqb_r[...]
        for (wv, bv, wo, bo, G1, B1, w1, b1, w2, b2, G2, B2) in (
                (wv0_r, bv0_r, wo0_r, bo0_r, g10_r, beta10_r, w10_r,
                 b10_r, w20_r, b20_r, g20_r, beta20_r),
                (wv1_r, bv1_r, wo1_r, bo1_r, g11_r, beta11_r, w11_r,
                 b11_r, w21_r, b21_r, g21_r, beta21_r)):
            v = _mmT(h, wv[...]) + bv[...]
            a = _mmT(v, wo[...]) + bo[...]
            h = _ln(h + a, G1[...], B1[...])
            ff = jnp.maximum(_mmT(h, w1[...]) + b1[...], 0.0)
            ff = _mmT(ff, w2[...]) + b2[...]
            h = _ln(h + ff, G2[...], B2[...])
        sf = jnp.mean(h, axis=0, keepdims=True)
        c_ref[...] = _mmT(sf, fusWs_r[...]) + fusb_r[...]

    return pl.pallas_call(
        body,
        out_shape=jax.ShapeDtypeStruct((1, H), jnp.float32),
    )(emb, seqW, seqb, wv0, bv0, wo0, bo0, g10, beta10, w10, b10, w20,
      b20, g20, beta20, wv1, bv1, wo1, bo1, g11, beta11, w11, b11, w21,
      b21, g21, beta21, fusWs, fusb)


def _final(x, s_all, cnt, c_row, gng, gnbeta, fusWm,
           resW, resb, resg, resbeta):
    n = x.shape[0]

    def body(x_r, s_r, cn_r, c_r, gng_r, gnbeta_r,
             fusWm_r, resW_r, resb_r, resg_r, resbeta_r, out_ref):
        sv = s_r[...]
        s = jnp.concatenate([sv[q] for q in range(4)], axis=-1)
        cv = cn_r[...]
        cnt = cv[0][:, 0:1] + cv[1][:, 0:1]
        x3 = x_r[...] + s / jnp.maximum(cnt, 1.0)
        xg = _ln(x3, gng_r[...], gnbeta_r[...])
        y = _mmT(xg, fusWm_r[...]) + c_r[...]
        z = _mmT(y, resW_r[...]) + resb_r[...]
        z = jnp.where(z != z, 0.0,
                      jnp.where(z == jnp.inf, 1e5,
                                jnp.where(z == -jnp.inf, -1e5, z)))
        z = _ln(z, resg_r[...], resbeta_r[...])
        z = jnp.clip(z, -10.0, 10.0)
        z = _gelu(z)
        out_ref[...] = y + z

    return pl.pallas_call(
        body,
        grid=(n // BN,),
        in_specs=[_rows((BN, H)),
                  pl.BlockSpec((4, BN, 128), lambda i: (0, i, 0)),
                  pl.BlockSpec((2, BN, 128), lambda i: (0, i, 0)),
                  _full((1, H)), _full((1, H)),
                  _full((1, H)), _full((H, H)), _full((H, H)), _full((1, H)),
                  _full((1, H)), _full((1, H))],
        out_specs=_rows((BN, H)),
        out_shape=jax.ShapeDtypeStruct((n, H), jnp.float32),
    )(x, s_all, cnt, c_row, gng, gnbeta, fusWm, resW,
      resb, resg, resbeta)


def kernel(mol_x, mol_edge_attr, mol_dist, mol_theta, mol_phi, mol_tau,
           mol_embedding, mol_edge_index, params):
    p = params
    n = mol_x.shape[0]
    row = mol_edge_index[0].astype(jnp.int32)
    col = mol_edge_index[1].astype(jnp.int32)

    def r2(v):
        return v.reshape(1, -1)

    mol_x_pad = jnp.pad(mol_x, ((0, 0), (0, 80 - mol_x.shape[1])))
    aW = jnp.pad(p['atom_W'], ((0, 0), (0, 80 - p['atom_W'].shape[1])))
    scal = jnp.stack([mol_dist, mol_theta, mol_phi, mol_tau], axis=1)
    scal = jnp.pad(scal, ((0, 0), (0, 4)))
    s4 = jnp.pad(p['spat_W'][:, :4], ((0, 0), (0, 4)))
    sb_w = p['spat_W'][:, 4:]

    w1a = [p[f'l{i}_W1'][:, :H] for i in range(3)]
    w1b = [p[f'l{i}_W1'][:, H:] for i in range(3)]
    w2 = [p[f'l{i}_W2'] for i in range(3)]

    x0, xa = _node_encoder(
        mol_x_pad, aW, r2(p['atom_b']), r2(p['atom_g']), r2(p['atom_beta']),
        p['np_W'], r2(p['np_b']), r2(p['np_g']), r2(p['np_beta']),
        w1a[0])

    cnt = _sc_counts(col, n)

    emb = mol_embedding.reshape(mol_embedding.shape[1], mol_embedding.shape[2])
    tw = []
    for i in range(2):
        win, bin_ = p[f't{i}_Win'], p[f't{i}_bin']
        tw += [win[2 * H:3 * H], r2(bin_[2 * H:3 * H]),
               p[f't{i}_Wout'], r2(p[f't{i}_bout']),
               r2(p[f't{i}_g1']), r2(p[f't{i}_beta1']),
               p[f't{i}_W1'], r2(p[f't{i}_b1']),
               p[f't{i}_W2'], r2(p[f't{i}_b2']),
               r2(p[f't{i}_g2']), r2(p[f't{i}_beta2'])]
    c_row = _seq_branch(emb, p['seq_W'], r2(p['seq_b']), *tw,
                        p['fus_W'][:, H:], r2(p['fus_b']))

    ea = None
    x = x0
    for i in range(3):
        g = _sc_gather(xa, row)
        if i == 0:
            ea, m_all = _edge_mlp0(
                mol_edge_attr, scal, g,
                p['bond_W'], r2(p['bond_b']), r2(p['bond_g']),
                r2(p['bond_beta']),
                s4, sb_w, r2(p['spat_b']), r2(p['spat_g']), r2(p['spat_beta']),
                p['ep_W'], r2(p['ep_b']), r2(p['ep_g']), r2(p['ep_beta']),
                w1b[0], r2(p['l0_b1']), r2(p['l0_g1']), r2(p['l0_beta1']),
                w2[0], r2(p['l0_b2']), r2(p['l0_g2']), r2(p['l0_beta2']))
        else:
            m_all = _edge_mlp(
                ea, g, w1b[i], r2(p[f'l{i}_b1']), r2(p[f'l{i}_g1']),
                r2(p[f'l{i}_beta1']), w2[i], r2(p[f'l{i}_b2']),
                r2(p[f'l{i}_g2']), r2(p[f'l{i}_beta2']))
        s_all = _sc_scatter_add(m_all, col, n)
        if i < 2:
            x, xa = _x_update(x, s_all, cnt, w1a[i + 1])

    out = _final(x, s_all, cnt, c_row,
                 r2(p['gn_g']), r2(p['gn_beta']), p['fus_W'][:, :H],
                 p['res_W'], r2(p['res_b']), r2(p['res_g']),
                 r2(p['res_beta']))
    return out


# speedup vs baseline: 1.1137x; 1.1137x over previous
import functools

import jax
import jax.numpy as jnp
from jax import lax
from jax.experimental import pallas as pl
from jax.experimental.pallas import tpu as pltpu
from jax.experimental.pallas import tpu_sc as plsc

N_NODES = 10000
N_EDGES = 160000
H = 512

NC, NS = 2, 16
NW = NC * NS

_mesh = functools.partial(
    plsc.VectorSubcoreMesh, core_axis_name="c", subcore_axis_name="s",
    num_cores=NC, num_subcores=NS)


def _gelu(x):
    return x * 0.5 * (1.0 + lax.erf(x * 0.7071067811865476))


def _ln(x, g, b):
    m = jnp.mean(x, axis=-1, keepdims=True)
    v = jnp.mean((x - m) ** 2, axis=-1, keepdims=True)
    return (x - m) * lax.rsqrt(v + 1e-5) * g + b


def _mmT(a, w):
    return lax.dot_general(a, w, (((1,), (1,)), ((), ())),
                           preferred_element_type=jnp.float32)


def _sc_gather(table, idx):
    V, D = table.shape
    E = idx.shape[0]
    per_w = E // NW
    assert per_w % 8 == 0
    C = 64
    NB = 3
    n_full = per_w // C
    n_tri = (n_full + NB - 1) // NB
    rem = per_w - n_full * C

    def body(table_hbm, idx_hbm, out_hbm, idx_all, r0, r1, r2,
             g0, g1, g2, w0, w1, w2):
        wid = lax.axis_index("s") * NC + lax.axis_index("c")
        base = wid * per_w
        rows = (r0, r1, r2)
        gsem = (g0, g1, g2)
        wsem = (w0, w1, w2)

        pltpu.sync_copy(idx_hbm.at[pl.ds(base, per_w)], idx_all)

        def gd(j, b):
            return pltpu.make_async_copy(
                table_hbm.at[idx_all.at[pl.ds(j * C, C)]], rows[b], gsem[b])

        def wd(j, b):
            return pltpu.make_async_copy(
                rows[b], out_hbm.at[pl.ds(base + j * C, C)], wsem[b])

        for b in range(min(NB, n_full)):
            gd(b, b).start()

        def tri(i, carry):
            for b in range(NB):
                j = i * NB + b

                @pl.when(j < n_full)
                def _():
                    gd(j, b).wait()
                    wd(j, b).start()
                    jn = j + NB

                    @pl.when(jn < n_full)
                    def _():
                        wd(j, b).wait()
                        gd(jn, b).start()

            return carry

        lax.fori_loop(0, n_tri, tri, 0)
        for b in range(min(NB, n_full)):
            jl = ((n_full - 1 - b) // NB) * NB + b
            wd(jl, b).wait()
        if rem:
            off = base + n_full * C
            pltpu.async_copy(table_hbm.at[idx_all.at[pl.ds(n_full * C, rem)]],
                             rows[0].at[pl.ds(0, rem)], gsem[0]).wait()
            pltpu.sync_copy(rows[0].at[pl.ds(0, rem)],
                            out_hbm.at[pl.ds(off, rem)])

    return pl.kernel(
        body,
        out_type=jax.ShapeDtypeStruct((E, D), jnp.float32),
        mesh=_mesh(),
        scratch_types=[
            pltpu.VMEM((per_w,), jnp.int32),
        ] + [pltpu.VMEM((C, D), jnp.float32)] * 3
        + [pltpu.SemaphoreType.DMA] * 6,
    )(table, idx)


def _sc_scatter_add(m2_chunks, col, n_nodes):
    E = col.shape[0]
    F = 128
    per_t = E // NS
    C = 128
    n_full = per_t // C
    rem = per_t - n_full * C
    SL = 64
    vp = ((n_nodes + NS * SL - 1) // (NS * SL)) * (NS * SL)
    rows_t = vp // NS
    n_sl = rows_t // SL

    def body(m_all, col_hbm, out_hbm,
             accum, zbuf, mb0, mb1, ib0, ib1, irem,
             is0, is1, ms0, ms1, as0, as1):
        c = lax.axis_index("c")
        s = lax.axis_index("s")
        mbuf = (mb0, mb1)
        ibuf = (ib0, ib1)
        isem = (is0, is1)
        msem = (ms0, ms1)
        asem = (as0, as1)

        zv = jnp.zeros((16,), jnp.float32)

        def zrow(i, carry):
            for j in range(F // 16):
                zbuf[i, pl.ds(j * 16, 16)] = zv
            return carry

        lax.fori_loop(0, SL, zrow, 0)

        for k in range(2):
            q = 2 * c + k

            def zslice(t, carry):
                pltpu.sync_copy(zbuf, accum.at[pl.ds(s * rows_t + t * SL, SL)])
                return carry

            lax.fori_loop(0, n_sl, zslice, 0)
            plsc.subcore_barrier()

            ebase = s * per_t

            def ld(j, b):
                off = ebase + j * C
                return (pltpu.make_async_copy(
                            col_hbm.at[pl.ds(off, C)], ibuf[b], isem[b]),
                        pltpu.make_async_copy(
                            m_all.at[q, pl.ds(off, C)], mbuf[b], msem[b]))

            def ad(b):
                return pltpu.make_async_copy(
                    mbuf[b], accum.at[ibuf[b]], asem[b])

            for b in range(2):
                di, dm = ld(b, b)
                di.start()
                dm.start()

            def step(i, carry):
                for b in range(2):
                    j = 2 * i + b

                    @pl.when(j < n_full)
                    def _():
                        di, dm = ld(j, b)
                        di.wait()
                        dm.wait()
                        ad(b).start(add=True)
                        jn = j + 2

                        @pl.when(jn < n_full)
                        def _():
                            ad(b).wait()
                            di2, dm2 = ld(jn, b)
                            di2.start()
                            dm2.start()

                return carry

            lax.fori_loop(0, (n_full + 1) // 2, step, 0)
            for b in range(min(2, n_full)):
                ad(b).wait()
            if rem:
                off = ebase + n_full * C
                pltpu.sync_copy(col_hbm.at[pl.ds(off, rem)], irem)
                pltpu.sync_copy(m_all.at[q, pl.ds(off, rem)],
                                mb0.at[pl.ds(0, rem)])
                pltpu.sync_copy(mb0.at[pl.ds(0, rem)], accum.at[irem],
                                add=True)

            plsc.subcore_barrier()
            def wslice(t, carry):
                off = s * rows_t + t * SL
                pltpu.sync_copy(accum.at[pl.ds(off, SL)], zbuf)
                pltpu.sync_copy(zbuf, out_hbm.at[q, pl.ds(off, SL)])
                return carry

            lax.fori_loop(0, n_sl, wslice, 0)
            lax.fori_loop(0, SL, zrow, 0)
            plsc.subcore_barrier()

    return pl.kernel(
        body,
        out_type=jax.ShapeDtypeStruct((4, vp, F), jnp.float32),
        mesh=_mesh(),
        scratch_types=[
            pltpu.VMEM_SHARED((vp, F), jnp.float32),
            pltpu.VMEM((SL, F), jnp.float32),
            pltpu.VMEM((C, F), jnp.float32),
            pltpu.VMEM((C, F), jnp.float32),
            pltpu.VMEM((C,), jnp.int32),
            pltpu.VMEM((C,), jnp.int32),
            pltpu.VMEM((max(rem, 8),), jnp.int32),
        ] + [pltpu.SemaphoreType.DMA] * 6,
    )(m2_chunks, col)


def _sc_counts(col, n_nodes):
    E = col.shape[0]
    F = 128
    per_t = E // (NC * NS)
    C = 128
    n_full = per_t // C
    rem = per_t - n_full * C
    SL = 128
    vp = ((n_nodes + NS * SL - 1) // (NS * SL)) * (NS * SL)
    rows_t = vp // NS
    n_sl = rows_t // SL

    def body(col_hbm, out_hbm, accum, zbuf, ones_v, ib0, ib1, irem,
             is0, is1, as0, as1):
        c = lax.axis_index("c")
        s = lax.axis_index("s")
        ibuf = (ib0, ib1)
        isem = (is0, is1)
        asem = (as0, as1)
        zv = jnp.zeros((16,), jnp.float32)
        ov = jnp.ones((16,), jnp.float32)

        def zrow(i, carry):
            for j in range(F // 16):
                zbuf[i, pl.ds(j * 16, 16)] = zv
            return carry

        lax.fori_loop(0, SL, zrow, 0)

        def orow(i, carry):
            for j in range(F // 16):
                ones_v[i, pl.ds(j * 16, 16)] = ov
            return carry

        lax.fori_loop(0, C, orow, 0)

        def zslice(t, carry):
            pltpu.sync_copy(zbuf, accum.at[pl.ds(s * rows_t + t * SL, SL)])
            return carry

        lax.fori_loop(0, n_sl, zslice, 0)
        plsc.subcore_barrier()

        ebase = (c * NS + s) * per_t

        def ld(j, b):
            return pltpu.make_async_copy(
                col_hbm.at[pl.ds(ebase + j * C, C)], ibuf[b], isem[b])

        def ad(b):
            return pltpu.make_async_copy(ones_v, accum.at[ibuf[b]], asem[b])

        for b in range(2):
            ld(b, b).start()

        def step(i, carry):
            for b in range(2):
                j = 2 * i + b
                ld(j, b).wait()
                ad(b).start(add=True)
                jn = j + 2

                @pl.when(jn < n_full)
                def _():
                    ad(b).wait()
                    ld(jn, b).start()

            return carry

        lax.fori_loop(0, n_full // 2, step, 0)
        if n_full % 2:
            ld(n_full - 1, 0).wait()
            ad(0).start(add=True)
        for b in range(2):
            ad(b).wait()
        if rem:
            off = ebase + n_full * C
            pltpu.sync_copy(col_hbm.at[pl.ds(off, rem)], irem)
            pltpu.sync_copy(ones_v.at[pl.ds(0, rem)], accum.at[irem],
                            add=True)
        plsc.subcore_barrier()

        def wslice(t, carry):
            off = s * rows_t + t * SL
            pltpu.sync_copy(accum.at[pl.ds(off, SL)], zbuf)
            pltpu.sync_copy(zbuf, out_hbm.at[c, pl.ds(off, SL)])
            return carry

        lax.fori_loop(0, n_sl, wslice, 0)

    return pl.kernel(
        body,
        out_type=jax.ShapeDtypeStruct((NC, vp, F), jnp.float32),
        mesh=_mesh(),
        scratch_types=[
            pltpu.VMEM_SHARED((vp, F), jnp.float32),
            pltpu.VMEM((SL, F), jnp.float32),
            pltpu.VMEM((C, F), jnp.float32),
            pltpu.VMEM((C,), jnp.int32),
            pltpu.VMEM((C,), jnp.int32),
            pltpu.VMEM((max(rem, 8),), jnp.int32),
        ] + [pltpu.SemaphoreType.DMA] * 4,
    )(col)


BN = 2000
BE = 1600
E_SPLIT = 76800


def _full(shape):
    return pl.BlockSpec(shape, lambda i: (0,) * len(shape))


def _rows(block):
    return pl.BlockSpec(block, lambda i: (i,) + (0,) * (len(block) - 1))


def _node_encoder(mol_x_pad, aW, ab, ag, abeta, npW, npb, npg, npbeta,
                  w1a0):
    n = mol_x_pad.shape[0]
    K = mol_x_pad.shape[1]

    def body(x_ref, aW_r, ab_r, ag_r, abeta_r, npW_r, npb_r, npg_r,
             npbeta_r, w1a0_r, x0_ref, xa0_ref):
        a = _mmT(x_ref[...], aW_r[...]) + ab_r[...]
        a = _gelu(_ln(a, ag_r[...], abeta_r[...]))
        x0 = _mmT(a, npW_r[...]) + npb_r[...]
        x0 = _gelu(_ln(x0, npg_r[...], npbeta_r[...]))
        x0_ref[...] = x0
        xa0_ref[...] = _mmT(x0, w1a0_r[...])

    return pl.pallas_call(
        body,
        grid=(n // BN,),
        in_specs=[
            _rows((BN, K)),
            _full((H // 2, K)), _full((1, H // 2)), _full((1, H // 2)),
            _full((1, H // 2)),
            _full((H, H // 2)), _full((1, H)), _full((1, H)), _full((1, H)),
            _full((H, H)),
        ],
        out_specs=[_rows((BN, H)), _rows((BN, H))],
        out_shape=[jax.ShapeDtypeStruct((n, H), jnp.float32),
                   jax.ShapeDtypeStruct((n, H), jnp.float32)],
    )(mol_x_pad, aW, ab, ag, abeta, npW, npb, npg, npbeta, w1a0)


def _edge_mlp0(attr, scal, g0, bW, bb, bg, bbeta, s4, sb_w, sb, sg, sbeta,
               epW, epb, epg, epbeta, w1b, b1, g1, beta1, w2, b2, g2,
               beta2):
    e = attr.shape[0]

    def body(attr_r, scal_r, g_r, bW_r, bb_r, bg_r, bbeta_r, s4_r, sbw_r,
             sb_r, sg_r, sbeta_r, epW_r, epb_r, epg_r, epbeta_r, w1b_r,
             b1_r, g1_r, beta1_r, w2_r, b2_r, g2_r, beta2_r,
             ea_ref, m_ref):
        bond = _mmT(attr_r[...], bW_r[...]) + bb_r[...]
        bond = _gelu(_ln(bond, bg_r[...], bbeta_r[...]))
        spat = (_mmT(scal_r[...], s4_r[...])
                + _mmT(bond, sbw_r[...]) + sb_r[...])
        spat = _gelu(_ln(spat, sg_r[...], sbeta_r[...]))
        ea = _mmT(spat, epW_r[...]) + epb_r[...]
        ea = _gelu(_ln(ea, epg_r[...], epbeta_r[...]))
        ea_ref[...] = ea
        t = g_r[...] + _mmT(ea, w1b_r[...]) + b1_r[...]
        t = _gelu(_ln(t, g1_r[...], beta1_r[...]))
        m = _mmT(t, w2_r[...]) + b2_r[...]
        m = _gelu(_ln(m, g2_r[...], beta2_r[...]))
        for q in range(4):
            m_ref[q] = m[:, q * 128:(q + 1) * 128]

    return pl.pallas_call(
        body,
        grid=(e // BE,),
        in_specs=[
            _rows((BE, 8)), _rows((BE, 8)), _rows((BE, H)),
            _full((128, 8)), _full((1, 128)), _full((1, 128)),
            _full((1, 128)),
            _full((256, 8)), _full((256, 128)), _full((1, 256)),
            _full((1, 256)), _full((1, 256)),
            _full((H, 256)), _full((1, H)), _full((1, H)), _full((1, H)),
            _full((H, H)), _full((1, H)), _full((1, H)), _full((1, H)),
            _full((H, H)), _full((1, H)), _full((1, H)), _full((1, H)),
        ],
        out_specs=[_rows((BE, H)),
                   pl.BlockSpec((4, BE, 128), lambda i: (0, i, 0))],
        out_shape=[jax.ShapeDtypeStruct((e, H), jnp.float32),
                   jax.ShapeDtypeStruct((4, e, 128), jnp.float32)],
    )(attr, scal, g0, bW, bb, bg, bbeta, s4, sb_w, sb, sg, sbeta, epW,
      epb, epg, epbeta, w1b, b1, g1, beta1, w2, b2, g2, beta2)


def _edge_mlp(ea, g, w1b, b1, g1, beta1, w2, b2, g2, beta2):
    e = ea.shape[0]

    def body(ea_r, g_r, w1b_r, b1_r, g1_r, beta1_r, w2_r, b2_r, g2_r,
             beta2_r, m_ref):
        t = g_r[...] + _mmT(ea_r[...], w1b_r[...]) + b1_r[...]
        t = _gelu(_ln(t, g1_r[...], beta1_r[...]))
        m = _mmT(t, w2_r[...]) + b2_r[...]
        m = _gelu(_ln(m, g2_r[...], beta2_r[...]))
        for q in range(4):
            m_ref[q] = m[:, q * 128:(q + 1) * 128]

    return pl.pallas_call(
        body,
        grid=(e // BE,),
        in_specs=[
            _rows((BE, H)), _rows((BE, H)),
            _full((H, H)), _full((1, H)), _full((1, H)), _full((1, H)),
            _full((H, H)), _full((1, H)), _full((1, H)), _full((1, H)),
        ],
        out_specs=pl.BlockSpec((4, BE, 128), lambda i: (0, i, 0)),
        out_shape=jax.ShapeDtypeStruct((4, e, 128), jnp.float32),
    )(ea, g, w1b, b1, g1, beta1, w2, b2, g2, beta2)


def _x_update(x, s_a, s_b, cnt, w1a):
    n = x.shape[0]

    def body(x_r, sa_r, sb_r, c_r, w1a_r, xn_ref, xa_ref):
        sva = sa_r[...]
        svb = sb_r[...]
        s = jnp.concatenate([sva[q] + svb[q] for q in range(4)], axis=-1)
        cv = c_r[...]
        cnt = cv[0][:, 0:1] + cv[1][:, 0:1]
        xn = x_r[...] + s / jnp.maximum(cnt, 1.0)
        xn_ref[...] = xn
        xa_ref[...] = _mmT(xn, w1a_r[...])

    return pl.pallas_call(
        body,
        grid=(n // BN,),
        in_specs=[_rows((BN, H)),
                  pl.BlockSpec((4, BN, 128), lambda i: (0, i, 0)),
                  pl.BlockSpec((4, BN, 128), lambda i: (0, i, 0)),
                  pl.BlockSpec((2, BN, 128), lambda i: (0, i, 0)),
                  _full((H, H))],
        out_specs=[_rows((BN, H)), _rows((BN, H))],
        out_shape=[jax.ShapeDtypeStruct((n, H), jnp.float32),
                   jax.ShapeDtypeStruct((n, H), jnp.float32)],
    )(x, s_a, s_b, cnt, w1a)


def _seq_branch(emb, seqW, seqb, wv0, bv0, wo0, bo0, g10, beta10, w10,
                b10, w20, b20, g20, beta20, wv1, bv1, wo1, bo1, g11,
                beta11, w11, b11, w21, b21, g21, beta21, fusWs, fusb):
    def body(emb_r, seqW_r, seqb_r, wv0_r, bv0_r, wo0_r, bo0_r, g10_r,
             beta10_r, w10_r, b10_r, w20_r, b20_r, g20_r, beta20_r,
             wv1_r, bv1_r, wo1_r, bo1_r, g11_r, beta11_r, w11_r, b11_r,
             w21_r, b21_r, g21_r, beta21_r, fusWs_r, fusb_r, c_ref):
        h = _mmT(emb_r[...], seqW_r[...]) + seqb_r[...]
        for (wv, bv, wo, bo, G1, B1, w1, b1, w2, b2, G2, B2) in (
                (wv0_r, bv0_r, wo0_r, bo0_r, g10_r, beta10_r, w10_r,
                 b10_r, w20_r, b20_r, g20_r, beta20_r),
                (wv1_r, bv1_r, wo1_r, bo1_r, g11_r, beta11_r, w11_r,
                 b11_r, w21_r, b21_r, g21_r, beta21_r)):
            v = _mmT(h, wv[...]) + bv[...]
            a = _mmT(v, wo[...]) + bo[...]
            h = _ln(h + a, G1[...], B1[...])
            ff = jnp.maximum(_mmT(h, w1[...]) + b1[...], 0.0)
            ff = _mmT(ff, w2[...]) + b2[...]
            h = _ln(h + ff, G2[...], B2[...])
        sf = jnp.mean(h, axis=0, keepdims=True)
        c_ref[...] = _mmT(sf, fusWs_r[...]) + fusb_r[...]

    return pl.pallas_call(
        body,
        out_shape=jax.ShapeDtypeStruct((1, H), jnp.float32),
    )(emb, seqW, seqb, wv0, bv0, wo0, bo0, g10, beta10, w10, b10, w20,
      b20, g20, beta20, wv1, bv1, wo1, bo1, g11, beta11, w11, b11, w21,
      b21, g21, beta21, fusWs, fusb)


def _final(x, s_a, s_b, cnt, c_row, gng, gnbeta, fusWm,
           resW, resb, resg, resbeta):
    n = x.shape[0]

    def body(x_r, sa_r, sb_r, cn_r, c_r, gng_r, gnbeta_r,
             fusWm_r, resW_r, resb_r, resg_r, resbeta_r, out_ref):
        sva = sa_r[...]
        svb = sb_r[...]
        s = jnp.concatenate([sva[q] + svb[q] for q in range(4)], axis=-1)
        cv = cn_r[...]
        cnt = cv[0][:, 0:1] + cv[1][:, 0:1]
        x3 = x_r[...] + s / jnp.maximum(cnt, 1.0)
        xg = _ln(x3, gng_r[...], gnbeta_r[...])
        y = _mmT(xg, fusWm_r[...]) + c_r[...]
        z = _mmT(y, resW_r[...]) + resb_r[...]
        z = jnp.where(z != z, 0.0,
                      jnp.where(z == jnp.inf, 1e5,
                                jnp.where(z == -jnp.inf, -1e5, z)))
        z = _ln(z, resg_r[...], resbeta_r[...])
        z = jnp.clip(z, -10.0, 10.0)
        z = _gelu(z)
        out_ref[...] = y + z

    return pl.pallas_call(
        body,
        grid=(n // BN,),
        in_specs=[_rows((BN, H)),
                  pl.BlockSpec((4, BN, 128), lambda i: (0, i, 0)),
                  pl.BlockSpec((4, BN, 128), lambda i: (0, i, 0)),
                  pl.BlockSpec((2, BN, 128), lambda i: (0, i, 0)),
                  _full((1, H)), _full((1, H)),
                  _full((1, H)), _full((H, H)), _full((H, H)), _full((1, H)),
                  _full((1, H)), _full((1, H))],
        out_specs=_rows((BN, H)),
        out_shape=jax.ShapeDtypeStruct((n, H), jnp.float32),
    )(x, s_a, s_b, cnt, c_row, gng, gnbeta, fusWm, resW,
      resb, resg, resbeta)


def kernel(mol_x, mol_edge_attr, mol_dist, mol_theta, mol_phi, mol_tau,
           mol_embedding, mol_edge_index, params):
    p = params
    n = mol_x.shape[0]
    row = mol_edge_index[0].astype(jnp.int32)
    col = mol_edge_index[1].astype(jnp.int32)

    def r2(v):
        return v.reshape(1, -1)

    mol_x_pad = jnp.pad(mol_x, ((0, 0), (0, 80 - mol_x.shape[1])))
    aW = jnp.pad(p['atom_W'], ((0, 0), (0, 80 - p['atom_W'].shape[1])))
    scal = jnp.stack([mol_dist, mol_theta, mol_phi, mol_tau], axis=1)
    scal = jnp.pad(scal, ((0, 0), (0, 4)))
    s4 = jnp.pad(p['spat_W'][:, :4], ((0, 0), (0, 4)))
    sb_w = p['spat_W'][:, 4:]

    w1a = [p[f'l{i}_W1'][:, :H] for i in range(3)]
    w1b = [p[f'l{i}_W1'][:, H:] for i in range(3)]
    w2 = [p[f'l{i}_W2'] for i in range(3)]

    x0, xa = _node_encoder(
        mol_x_pad, aW, r2(p['atom_b']), r2(p['atom_g']), r2(p['atom_beta']),
        p['np_W'], r2(p['np_b']), r2(p['np_g']), r2(p['np_beta']),
        w1a[0])

    cnt = _sc_counts(col, n)

    emb = mol_embedding.reshape(mol_embedding.shape[1], mol_embedding.shape[2])
    tw = []
    for i in range(2):
        win, bin_ = p[f't{i}_Win'], p[f't{i}_bin']
        tw += [win[2 * H:3 * H], r2(bin_[2 * H:3 * H]),
               p[f't{i}_Wout'], r2(p[f't{i}_bout']),
               r2(p[f't{i}_g1']), r2(p[f't{i}_beta1']),
               p[f't{i}_W1'], r2(p[f't{i}_b1']),
               p[f't{i}_W2'], r2(p[f't{i}_b2']),
               r2(p[f't{i}_g2']), r2(p[f't{i}_beta2'])]
    c_row = _seq_branch(emb, p['seq_W'], r2(p['seq_b']), *tw,
                        p['fus_W'][:, H:], r2(p['fus_b']))

    SP = E_SPLIT
    row_h = (row[:SP], row[SP:])
    col_h = (col[:SP], col[SP:])
    attr_h = (mol_edge_attr[:SP], mol_edge_attr[SP:])
    scal_h = (scal[:SP], scal[SP:])

    ea_h = [None, None]
    x = x0
    for i in range(3):
        m_h = [None, None]
        for hh in range(2):
            g = _sc_gather(xa, row_h[hh])
            if i == 0:
                ea_h[hh], m_h[hh] = _edge_mlp0(
                    attr_h[hh], scal_h[hh], g,
                    p['bond_W'], r2(p['bond_b']), r2(p['bond_g']),
                    r2(p['bond_beta']),
                    s4, sb_w, r2(p['spat_b']), r2(p['spat_g']),
                    r2(p['spat_beta']),
                    p['ep_W'], r2(p['ep_b']), r2(p['ep_g']), r2(p['ep_beta']),
                    w1b[0], r2(p['l0_b1']), r2(p['l0_g1']), r2(p['l0_beta1']),
                    w2[0], r2(p['l0_b2']), r2(p['l0_g2']), r2(p['l0_beta2']))
            else:
                m_h[hh] = _edge_mlp(
                    ea_h[hh], g, w1b[i], r2(p[f'l{i}_b1']), r2(p[f'l{i}_g1']),
                    r2(p[f'l{i}_beta1']), w2[i], r2(p[f'l{i}_b2']),
                    r2(p[f'l{i}_g2']), r2(p[f'l{i}_beta2']))
        s_a = _sc_scatter_add(m_h[0], col_h[0], n)
        s_b = _sc_scatter_add(m_h[1], col_h[1], n)
        if i < 2:
            x, xa = _x_update(x, s_a, s_b, cnt, w1a[i + 1])

    out = _final(x, s_a, s_b, cnt, c_row,
                 r2(p['gn_g']), r2(p['gn_beta']), p['fus_W'][:, :H],
                 p['res_W'], r2(p['res_b']), r2(p['res_g']),
                 r2(p['res_beta']))
    return out


# speedup vs baseline: 1.1174x; 1.0033x over previous
import functools

import jax
import jax.numpy as jnp
from jax import lax
from jax.experimental import pallas as pl
from jax.experimental.pallas import tpu as pltpu
from jax.experimental.pallas import tpu_sc as plsc

N_NODES = 10000
N_EDGES = 160000
H = 512

NC, NS = 2, 16
NW = NC * NS

_mesh = functools.partial(
    plsc.VectorSubcoreMesh, core_axis_name="c", subcore_axis_name="s",
    num_cores=NC, num_subcores=NS)


def _gelu(x):
    return x * 0.5 * (1.0 + lax.erf(x * 0.7071067811865476))


def _ln(x, g, b):
    m = jnp.mean(x, axis=-1, keepdims=True)
    v = jnp.mean((x - m) ** 2, axis=-1, keepdims=True)
    return (x - m) * lax.rsqrt(v + 1e-5) * g + b


def _mmT(a, w):
    return lax.dot_general(a, w, (((1,), (1,)), ((), ())),
                           preferred_element_type=jnp.float32)


def _sc_gather(table, idx):
    V, D = table.shape
    E = idx.shape[0]
    per_w = E // NW
    assert per_w % 8 == 0
    C = 64
    NB = 3
    n_full = per_w // C
    n_tri = (n_full + NB - 1) // NB
    rem = per_w - n_full * C

    def body(table_hbm, idx_hbm, out_hbm, idx_all, r0, r1, r2,
             g0, g1, g2, w0, w1, w2):
        wid = lax.axis_index("s") * NC + lax.axis_index("c")
        base = wid * per_w
        rows = (r0, r1, r2)
        gsem = (g0, g1, g2)
        wsem = (w0, w1, w2)

        pltpu.sync_copy(idx_hbm.at[pl.ds(base, per_w)], idx_all)

        def gd(j, b):
            return pltpu.make_async_copy(
                table_hbm.at[idx_all.at[pl.ds(j * C, C)]], rows[b], gsem[b])

        def wd(j, b):
            return pltpu.make_async_copy(
                rows[b], out_hbm.at[pl.ds(base + j * C, C)], wsem[b])

        for b in range(min(NB, n_full)):
            gd(b, b).start()

        def tri(i, carry):
            for b in range(NB):
                j = i * NB + b

                @pl.when(j < n_full)
                def _():
                    gd(j, b).wait()
                    wd(j, b).start()
                    jn = j + NB

                    @pl.when(jn < n_full)
                    def _():
                        wd(j, b).wait()
                        gd(jn, b).start()

            return carry

        lax.fori_loop(0, n_tri, tri, 0)
        for b in range(min(NB, n_full)):
            jl = ((n_full - 1 - b) // NB) * NB + b
            wd(jl, b).wait()
        if rem:
            off = base + n_full * C
            pltpu.async_copy(table_hbm.at[idx_all.at[pl.ds(n_full * C, rem)]],
                             rows[0].at[pl.ds(0, rem)], gsem[0]).wait()
            pltpu.sync_copy(rows[0].at[pl.ds(0, rem)],
                            out_hbm.at[pl.ds(off, rem)])

    return pl.kernel(
        body,
        out_type=jax.ShapeDtypeStruct((E, D), jnp.float32),
        mesh=_mesh(),
        scratch_types=[
            pltpu.VMEM((per_w,), jnp.int32),
        ] + [pltpu.VMEM((C, D), jnp.float32)] * 3
        + [pltpu.SemaphoreType.DMA] * 6,
    )(table, idx)


def _sc_scatter_add(m2_chunks, col, n_nodes):
    E = col.shape[0]
    F = 128
    per_t = E // NS
    C = 128
    n_full = per_t // C
    rem = per_t - n_full * C
    SL = 64
    vp = ((n_nodes + NS * SL - 1) // (NS * SL)) * (NS * SL)
    rows_t = vp // NS
    n_sl = rows_t // SL

    def body(m_all, col_hbm, out_hbm,
             accum, zbuf, mb0, mb1, ib0, ib1, irem,
             is0, is1, ms0, ms1, as0, as1, zsem, wsem):
        c = lax.axis_index("c")
        s = lax.axis_index("s")
        mbuf = (mb0, mb1)
        ibuf = (ib0, ib1)
        isem = (is0, is1)
        msem = (ms0, ms1)
        asem = (as0, as1)

        zv = jnp.zeros((16,), jnp.float32)

        def zrow(i, carry):
            for j in range(F // 16):
                zbuf[i, pl.ds(j * 16, 16)] = zv
            return carry

        lax.fori_loop(0, SL, zrow, 0)

        for k in range(2):
            q = 2 * c + k

            def zd(t):
                return pltpu.make_async_copy(
                    zbuf, accum.at[pl.ds(s * rows_t + t * SL, SL)], zsem)

            def zfire(t, carry):
                zd(t).start()
                return carry

            def zdrain(t, carry):
                zd(t).wait()
                return carry

            lax.fori_loop(0, n_sl, zfire, 0)
            lax.fori_loop(0, n_sl, zdrain, 0)
            plsc.subcore_barrier()

            ebase = s * per_t

            def ld(j, b):
                off = ebase + j * C
                return (pltpu.make_async_copy(
                            col_hbm.at[pl.ds(off, C)], ibuf[b], isem[b]),
                        pltpu.make_async_copy(
                            m_all.at[q, pl.ds(off, C)], mbuf[b], msem[b]))

            def ad(b):
                return pltpu.make_async_copy(
                    mbuf[b], accum.at[ibuf[b]], asem[b])

            for b in range(2):
                di, dm = ld(b, b)
                di.start()
                dm.start()

            def step(i, carry):
                for b in range(2):
                    j = 2 * i + b

                    @pl.when(j < n_full)
                    def _():
                        di, dm = ld(j, b)
                        di.wait()
                        dm.wait()
                        ad(b).start(add=True)
                        jn = j + 2

                        @pl.when(jn < n_full)
                        def _():
                            ad(b).wait()
                            di2, dm2 = ld(jn, b)
                            di2.start()
                            dm2.start()

                return carry

            lax.fori_loop(0, (n_full + 1) // 2, step, 0)
            for b in range(min(2, n_full)):
                ad(b).wait()
            if rem:
                off = ebase + n_full * C
                pltpu.sync_copy(col_hbm.at[pl.ds(off, rem)], irem)
                pltpu.sync_copy(m_all.at[q, pl.ds(off, rem)],
                                mb0.at[pl.ds(0, rem)])
                pltpu.sync_copy(mb0.at[pl.ds(0, rem)], accum.at[irem],
                                add=True)

            plsc.subcore_barrier()

            def wd(t):
                off = s * rows_t + t * SL
                return pltpu.make_async_copy(
                    accum.at[pl.ds(off, SL)], out_hbm.at[q, pl.ds(off, SL)],
                    wsem)

            def wfire(t, carry):
                wd(t).start()
                return carry

            def wdrain(t, carry):
                wd(t).wait()
                return carry

            lax.fori_loop(0, n_sl, wfire, 0)
            lax.fori_loop(0, n_sl, wdrain, 0)
            plsc.subcore_barrier()

    return pl.kernel(
        body,
        out_type=jax.ShapeDtypeStruct((4, vp, F), jnp.float32),
        mesh=_mesh(),
        scratch_types=[
            pltpu.VMEM_SHARED((vp, F), jnp.float32),
            pltpu.VMEM((SL, F), jnp.float32),
            pltpu.VMEM((C, F), jnp.float32),
            pltpu.VMEM((C, F), jnp.float32),
            pltpu.VMEM((C,), jnp.int32),
            pltpu.VMEM((C,), jnp.int32),
            pltpu.VMEM((max(rem, 8),), jnp.int32),
        ] + [pltpu.SemaphoreType.DMA] * 8,
    )(m2_chunks, col)


def _sc_counts(col, n_nodes):
    E = col.shape[0]
    F = 128
    per_t = E // (NC * NS)
    C = 128
    n_full = per_t // C
    rem = per_t - n_full * C
    SL = 128
    vp = ((n_nodes + NS * SL - 1) // (NS * SL)) * (NS * SL)
    rows_t = vp // NS
    n_sl = rows_t // SL

    def body(col_hbm, out_hbm, accum, zbuf, ones_v, ib0, ib1, irem,
             is0, is1, as0, as1, zsem, wsem):
        c = lax.axis_index("c")
        s = lax.axis_index("s")
        ibuf = (ib0, ib1)
        isem = (is0, is1)
        asem = (as0, as1)
        zv = jnp.zeros((16,), jnp.float32)
        ov = jnp.ones((16,), jnp.float32)

        def zrow(i, carry):
            for j in range(F // 16):
                zbuf[i, pl.ds(j * 16, 16)] = zv
            return carry

        lax.fori_loop(0, SL, zrow, 0)

        def orow(i, carry):
            for j in range(F // 16):
                ones_v[i, pl.ds(j * 16, 16)] = ov
            return carry

        lax.fori_loop(0, C, orow, 0)

        def zd(t):
            return pltpu.make_async_copy(
                zbuf, accum.at[pl.ds(s * rows_t + t * SL, SL)], zsem)

        def zfire(t, carry):
            zd(t).start()
            return carry

        def zdrain(t, carry):
            zd(t).wait()
            return carry

        lax.fori_loop(0, n_sl, zfire, 0)
        lax.fori_loop(0, n_sl, zdrain, 0)
        plsc.subcore_barrier()

        ebase = (c * NS + s) * per_t

        def ld(j, b):
            return pltpu.make_async_copy(
                col_hbm.at[pl.ds(ebase + j * C, C)], ibuf[b], isem[b])

        def ad(b):
            return pltpu.make_async_copy(ones_v, accum.at[ibuf[b]], asem[b])

        for b in range(2):
            ld(b, b).start()

        def step(i, carry):
            for b in range(2):
                j = 2 * i + b
                ld(j, b).wait()
                ad(b).start(add=True)
                jn = j + 2

                @pl.when(jn < n_full)
                def _():
                    ad(b).wait()
                    ld(jn, b).start()

            return carry

        lax.fori_loop(0, n_full // 2, step, 0)
        if n_full % 2:
            ld(n_full - 1, 0).wait()
            ad(0).start(add=True)
        for b in range(2):
            ad(b).wait()
        if rem:
            off = ebase + n_full * C
            pltpu.sync_copy(col_hbm.at[pl.ds(off, rem)], irem)
            pltpu.sync_copy(ones_v.at[pl.ds(0, rem)], accum.at[irem],
                            add=True)
        plsc.subcore_barrier()

        def wd(t):
            off = s * rows_t + t * SL
            return pltpu.make_async_copy(
                accum.at[pl.ds(off, SL)], out_hbm.at[c, pl.ds(off, SL)],
                wsem)

        def wfire(t, carry):
            wd(t).start()
            return carry

        def wdrain(t, carry):
            wd(t).wait()
            return carry

        lax.fori_loop(0, n_sl, wfire, 0)
        lax.fori_loop(0, n_sl, wdrain, 0)

    return pl.kernel(
        body,
        out_type=jax.ShapeDtypeStruct((NC, vp, F), jnp.float32),
        mesh=_mesh(),
        scratch_types=[
            pltpu.VMEM_SHARED((vp, F), jnp.float32),
            pltpu.VMEM((SL, F), jnp.float32),
            pltpu.VMEM((C, F), jnp.float32),
            pltpu.VMEM((C,), jnp.int32),
            pltpu.VMEM((C,), jnp.int32),
            pltpu.VMEM((max(rem, 8),), jnp.int32),
        ] + [pltpu.SemaphoreType.DMA] * 6,
    )(col)


BN = 2000
BE = 1600
E_SPLIT = 76800


def _full(shape):
    return pl.BlockSpec(shape, lambda i: (0,) * len(shape))


def _rows(block):
    return pl.BlockSpec(block, lambda i: (i,) + (0,) * (len(block) - 1))


def _node_encoder(mol_x_pad, aW, ab, ag, abeta, npW, npb, npg, npbeta,
                  w1a0):
    n = mol_x_pad.shape[0]
    K = mol_x_pad.shape[1]

    def body(x_ref, aW_r, ab_r, ag_r, abeta_r, npW_r, npb_r, npg_r,
             npbeta_r, w1a0_r, x0_ref, xa0_ref):
        a = _mmT(x_ref[...], aW_r[...]) + ab_r[...]
        a = _gelu(_ln(a, ag_r[...], abeta_r[...]))
        x0 = _mmT(a, npW_r[...]) + npb_r[...]
        x0 = _gelu(_ln(x0, npg_r[...], npbeta_r[...]))
        x0_ref[...] = x0
        xa0_ref[...] = _mmT(x0, w1a0_r[...])

    return pl.pallas_call(
        body,
        grid=(n // BN,),
        in_specs=[
            _rows((BN, K)),
            _full((H // 2, K)), _full((1, H // 2)), _full((1, H // 2)),
            _full((1, H // 2)),
            _full((H, H // 2)), _full((1, H)), _full((1, H)), _full((1, H)),
            _full((H, H)),
        ],
        out_specs=[_rows((BN, H)), _rows((BN, H))],
        out_shape=[jax.ShapeDtypeStruct((n, H), jnp.float32),
                   jax.ShapeDtypeStruct((n, H), jnp.float32)],
    )(mol_x_pad, aW, ab, ag, abeta, npW, npb, npg, npbeta, w1a0)


def _edge_mlp0(attr, scal, g0, bW, bb, bg, bbeta, s4, sb_w, sb, sg, sbeta,
               epW, epb, epg, epbeta, w1b, b1, g1, beta1, w2, b2, g2,
               beta2):
    e = attr.shape[0]

    def body(attr_r, scal_r, g_r, bW_r, bb_r, bg_r, bbeta_r, s4_r, sbw_r,
             sb_r, sg_r, sbeta_r, epW_r, epb_r, epg_r, epbeta_r, w1b_r,
             b1_r, g1_r, beta1_r, w2_r, b2_r, g2_r, beta2_r,
             ea_ref, m_ref):
        bond = _mmT(attr_r[...], bW_r[...]) + bb_r[...]
        bond = _gelu(_ln(bond, bg_r[...], bbeta_r[...]))
        spat = (_mmT(scal_r[...], s4_r[...])
                + _mmT(bond, sbw_r[...]) + sb_r[...])
        spat = _gelu(_ln(spat, sg_r[...], sbeta_r[...]))
        ea = _mmT(spat, epW_r[...]) + epb_r[...]
        ea = _gelu(_ln(ea, epg_r[...], epbeta_r[...]))
        ea_ref[...] = ea
        t = g_r[...] + _mmT(ea, w1b_r[...]) + b1_r[...]
        t = _gelu(_ln(t, g1_r[...], beta1_r[...]))
        m = _mmT(t, w2_r[...]) + b2_r[...]
        m = _gelu(_ln(m, g2_r[...], beta2_r[...]))
        for q in range(4):
            m_ref[q] = m[:, q * 128:(q + 1) * 128]

    return pl.pallas_call(
        body,
        grid=(e // BE,),
        in_specs=[
            _rows((BE, 8)), _rows((BE, 8)), _rows((BE, H)),
            _full((128, 8)), _full((1, 128)), _full((1, 128)),
            _full((1, 128)),
            _full((256, 8)), _full((256, 128)), _full((1, 256)),
            _full((1, 256)), _full((1, 256)),
            _full((H, 256)), _full((1, H)), _full((1, H)), _full((1, H)),
            _full((H, H)), _full((1, H)), _full((1, H)), _full((1, H)),
            _full((H, H)), _full((1, H)), _full((1, H)), _full((1, H)),
        ],
        out_specs=[_rows((BE, H)),
                   pl.BlockSpec((4, BE, 128), lambda i: (0, i, 0))],
        out_shape=[jax.ShapeDtypeStruct((e, H), jnp.float32),
                   jax.ShapeDtypeStruct((4, e, 128), jnp.float32)],
    )(attr, scal, g0, bW, bb, bg, bbeta, s4, sb_w, sb, sg, sbeta, epW,
      epb, epg, epbeta, w1b, b1, g1, beta1, w2, b2, g2, beta2)


def _edge_mlp(ea, g, w1b, b1, g1, beta1, w2, b2, g2, beta2):
    e = ea.shape[0]

    def body(ea_r, g_r, w1b_r, b1_r, g1_r, beta1_r, w2_r, b2_r, g2_r,
             beta2_r, m_ref):
        t = g_r[...] + _mmT(ea_r[...], w1b_r[...]) + b1_r[...]
        t = _gelu(_ln(t, g1_r[...], beta1_r[...]))
        m = _mmT(t, w2_r[...]) + b2_r[...]
        m = _gelu(_ln(m, g2_r[...], beta2_r[...]))
        for q in range(4):
            m_ref[q] = m[:, q * 128:(q + 1) * 128]

    return pl.pallas_call(
        body,
        grid=(e // BE,),
        in_specs=[
            _rows((BE, H)), _rows((BE, H)),
            _full((H, H)), _full((1, H)), _full((1, H)), _full((1, H)),
            _full((H, H)), _full((1, H)), _full((1, H)), _full((1, H)),
        ],
        out_specs=pl.BlockSpec((4, BE, 128), lambda i: (0, i, 0)),
        out_shape=jax.ShapeDtypeStruct((4, e, 128), jnp.float32),
    )(ea, g, w1b, b1, g1, beta1, w2, b2, g2, beta2)


def _x_update(x, s_a, s_b, cnt, w1a):
    n = x.shape[0]

    def body(x_r, sa_r, sb_r, c_r, w1a_r, xn_ref, xa_ref):
        sva = sa_r[...]
        svb = sb_r[...]
        s = jnp.concatenate([sva[q] + svb[q] for q in range(4)], axis=-1)
        cv = c_r[...]
        cnt = cv[0][:, 0:1] + cv[1][:, 0:1]
        xn = x_r[...] + s / jnp.maximum(cnt, 1.0)
        xn_ref[...] = xn
        xa_ref[...] = _mmT(xn, w1a_r[...])

    return pl.pallas_call(
        body,
        grid=(n // BN,),
        in_specs=[_rows((BN, H)),
                  pl.BlockSpec((4, BN, 128), lambda i: (0, i, 0)),
                  pl.BlockSpec((4, BN, 128), lambda i: (0, i, 0)),
                  pl.BlockSpec((2, BN, 128), lambda i: (0, i, 0)),
                  _full((H, H))],
        out_specs=[_rows((BN, H)), _rows((BN, H))],
        out_shape=[jax.ShapeDtypeStruct((n, H), jnp.float32),
                   jax.ShapeDtypeStruct((n, H), jnp.float32)],
    )(x, s_a, s_b, cnt, w1a)


def _seq_branch(emb, seqW, seqb, wv0, bv0, wo0, bo0, g10, beta10, w10,
                b10, w20, b20, g20, beta20, wv1, bv1, wo1, bo1, g11,
                beta11, w11, b11, w21, b21, g21, beta21, fusWs, fusb):
    def body(emb_r, seqW_r, seqb_r, wv0_r, bv0_r, wo0_r, bo0_r, g10_r,
             beta10_r, w10_r, b10_r, w20_r, b20_r, g20_r, beta20_r,
             wv1_r, bv1_r, wo1_r, bo1_r, g11_r, beta11_r, w11_r, b11_r,
             w21_r, b21_r, g21_r, beta21_r, fusWs_r, fusb_r, c_ref):
        h = _mmT(emb_r[...], seqW_r[...]) + seqb_r[...]
        for (wv, bv, wo, bo, G1, B1, w1, b1, w2, b2, G2, B2) in (
                (wv0_r, bv0_r, wo0_r, bo0_r, g10_r, beta10_r, w10_r,
                 b10_r, w20_r, b20_r, g20_r, beta20_r),
                (wv1_r, bv1_r, wo1_r, bo1_r, g11_r, beta11_r, w11_r,
                 b11_r, w21_r, b21_r, g21_r, beta21_r)):
            v = _mmT(h, wv[...]) + bv[...]
            a = _mmT(v, wo[...]) + bo[...]
            h = _ln(h + a, G1[...], B1[...])
            ff = jnp.maximum(_mmT(h, w1[...]) + b1[...], 0.0)
            ff = _mmT(ff, w2[...]) + b2[...]
            h = _ln(h + ff, G2[...], B2[...])
        sf = jnp.mean(h, axis=0, keepdims=True)
        c_ref[...] = _mmT(sf, fusWs_r[...]) + fusb_r[...]

    return pl.pallas_call(
        body,
        out_shape=jax.ShapeDtypeStruct((1, H), jnp.float32),
    )(emb, seqW, seqb, wv0, bv0, wo0, bo0, g10, beta10, w10, b10, w20,
      b20, g20, beta20, wv1, bv1, wo1, bo1, g11, beta11, w11, b11, w21,
      b21, g21, beta21, fusWs, fusb)


def _final(x, s_a, s_b, cnt, c_row, gng, gnbeta, fusWm,
           resW, resb, resg, resbeta):
    n = x.shape[0]

    def body(x_r, sa_r, sb_r, cn_r, c_r, gng_r, gnbeta_r,
             fusWm_r, resW_r, resb_r, resg_r, resbeta_r, out_ref):
        sva = sa_r[...]
        svb = sb_r[...]
        s = jnp.concatenate([sva[q] + svb[q] for q in range(4)], axis=-1)
        cv = cn_r[...]
        cnt = cv[0][:, 0:1] + cv[1][:, 0:1]
        x3 = x_r[...] + s / jnp.maximum(cnt, 1.0)
        xg = _ln(x3, gng_r[...], gnbeta_r[...])
        y = _mmT(xg, fusWm_r[...]) + c_r[...]
        z = _mmT(y, resW_r[...]) + resb_r[...]
        z = jnp.where(z != z, 0.0,
                      jnp.where(z == jnp.inf, 1e5,
                                jnp.where(z == -jnp.inf, -1e5, z)))
        z = _ln(z, resg_r[...], resbeta_r[...])
        z = jnp.clip(z, -10.0, 10.0)
        z = _gelu(z)
        out_ref[...] = y + z

    return pl.pallas_call(
        body,
        grid=(n // BN,),
        in_specs=[_rows((BN, H)),
                  pl.BlockSpec((4, BN, 128), lambda i: (0, i, 0)),
                  pl.BlockSpec((4, BN, 128), lambda i: (0, i, 0)),
                  pl.BlockSpec((2, BN, 128), lambda i: (0, i, 0)),
                  _full((1, H)), _full((1, H)),
                  _full((1, H)), _full((H, H)), _full((H, H)), _full((1, H)),
                  _full((1, H)), _full((1, H))],
        out_specs=_rows((BN, H)),
        out_shape=jax.ShapeDtypeStruct((n, H), jnp.float32),
    )(x, s_a, s_b, cnt, c_row, gng, gnbeta, fusWm, resW,
      resb, resg, resbeta)


def kernel(mol_x, mol_edge_attr, mol_dist, mol_theta, mol_phi, mol_tau,
           mol_embedding, mol_edge_index, params):
    p = params
    n = mol_x.shape[0]
    row = mol_edge_index[0].astype(jnp.int32)
    col = mol_edge_index[1].astype(jnp.int32)

    def r2(v):
        return v.reshape(1, -1)

    mol_x_pad = jnp.pad(mol_x, ((0, 0), (0, 80 - mol_x.shape[1])))
    aW = jnp.pad(p['atom_W'], ((0, 0), (0, 80 - p['atom_W'].shape[1])))
    scal = jnp.stack([mol_dist, mol_theta, mol_phi, mol_tau], axis=1)
    scal = jnp.pad(scal, ((0, 0), (0, 4)))
    s4 = jnp.pad(p['spat_W'][:, :4], ((0, 0), (0, 4)))
    sb_w = p['spat_W'][:, 4:]

    w1a = [p[f'l{i}_W1'][:, :H] for i in range(3)]
    w1b = [p[f'l{i}_W1'][:, H:] for i in range(3)]
    w2 = [p[f'l{i}_W2'] for i in range(3)]

    x0, xa = _node_encoder(
        mol_x_pad, aW, r2(p['atom_b']), r2(p['atom_g']), r2(p['atom_beta']),
        p['np_W'], r2(p['np_b']), r2(p['np_g']), r2(p['np_beta']),
        w1a[0])

    cnt = _sc_counts(col, n)

    emb = mol_embedding.reshape(mol_embedding.shape[1], mol_embedding.shape[2])
    tw = []
    for i in range(2):
        win, bin_ = p[f't{i}_Win'], p[f't{i}_bin']
        tw += [win[2 * H:3 * H], r2(bin_[2 * H:3 * H]),
               p[f't{i}_Wout'], r2(p[f't{i}_bout']),
               r2(p[f't{i}_g1']), r2(p[f't{i}_beta1']),
               p[f't{i}_W1'], r2(p[f't{i}_b1']),
               p[f't{i}_W2'], r2(p[f't{i}_b2']),
               r2(p[f't{i}_g2']), r2(p[f't{i}_beta2'])]
    c_row = _seq_branch(emb, p['seq_W'], r2(p['seq_b']), *tw,
                        p['fus_W'][:, H:], r2(p['fus_b']))

    SP = E_SPLIT
    row_h = (row[:SP], row[SP:])
    col_h = (col[:SP], col[SP:])
    attr_h = (mol_edge_attr[:SP], mol_edge_attr[SP:])
    scal_h = (scal[:SP], scal[SP:])

    ea_h = [None, None]
    x = x0
    for i in range(3):
        m_h = [None, None]
        for hh in range(2):
            g = _sc_gather(xa, row_h[hh])
            if i == 0:
                ea_h[hh], m_h[hh] = _edge_mlp0(
                    attr_h[hh], scal_h[hh], g,
                    p['bond_W'], r2(p['bond_b']), r2(p['bond_g']),
                    r2(p['bond_beta']),
                    s4, sb_w, r2(p['spat_b']), r2(p['spat_g']),
                    r2(p['spat_beta']),
                    p['ep_W'], r2(p['ep_b']), r2(p['ep_g']), r2(p['ep_beta']),
                    w1b[0], r2(p['l0_b1']), r2(p['l0_g1']), r2(p['l0_beta1']),
                    w2[0], r2(p['l0_b2']), r2(p['l0_g2']), r2(p['l0_beta2']))
            else:
                m_h[hh] = _edge_mlp(
                    ea_h[hh], g, w1b[i], r2(p[f'l{i}_b1']), r2(p[f'l{i}_g1']),
                    r2(p[f'l{i}_beta1']), w2[i], r2(p[f'l{i}_b2']),
                    r2(p[f'l{i}_g2']), r2(p[f'l{i}_beta2']))
        s_a = _sc_scatter_add(m_h[0], col_h[0], n)
        s_b = _sc_scatter_add(m_h[1], col_h[1], n)
        if i < 2:
            x, xa = _x_update(x, s_a, s_b, cnt, w1a[i + 1])

    out = _final(x, s_a, s_b, cnt, c_row,
                 r2(p['gn_g']), r2(p['gn_beta']), p['fus_W'][:, :H],
                 p['res_W'], r2(p['res_b']), r2(p['res_g']),
                 r2(p['res_beta']))
    return out


# speedup vs baseline: 1.1177x; 1.0003x over previous
import functools

import jax
import jax.numpy as jnp
from jax import lax
from jax.experimental import pallas as pl
from jax.experimental.pallas import tpu as pltpu
from jax.experimental.pallas import tpu_sc as plsc

N_NODES = 10000
N_EDGES = 160000
H = 512

NC, NS = 2, 16
NW = NC * NS

_mesh = functools.partial(
    plsc.VectorSubcoreMesh, core_axis_name="c", subcore_axis_name="s",
    num_cores=NC, num_subcores=NS)


def _gelu(x):
    return x * 0.5 * (1.0 + lax.erf(x * 0.7071067811865476))


def _ln(x, g, b):
    m = jnp.mean(x, axis=-1, keepdims=True)
    v = jnp.mean((x - m) ** 2, axis=-1, keepdims=True)
    return (x - m) * lax.rsqrt(v + 1e-5) * g + b


def _mmT(a, w):
    return lax.dot_general(a, w, (((1,), (1,)), ((), ())),
                           preferred_element_type=jnp.float32)


def _sc_gather(table, idx):
    V, D = table.shape
    E = idx.shape[0]
    per_w = E // NW
    assert per_w % 8 == 0
    C = 80
    NB = 3
    n_full = per_w // C
    n_tri = (n_full + NB - 1) // NB
    rem = per_w - n_full * C

    def body(table_hbm, idx_hbm, out_hbm, idx_all, r0, r1, r2,
             g0, g1, g2, w0, w1, w2):
        wid = lax.axis_index("s") * NC + lax.axis_index("c")
        base = wid * per_w
        rows = (r0, r1, r2)
        gsem = (g0, g1, g2)
        wsem = (w0, w1, w2)

        pltpu.sync_copy(idx_hbm.at[pl.ds(base, per_w)], idx_all)

        def gd(j, b):
            return pltpu.make_async_copy(
                table_hbm.at[idx_all.at[pl.ds(j * C, C)]], rows[b], gsem[b])

        def wd(j, b):
            return pltpu.make_async_copy(
                rows[b], out_hbm.at[pl.ds(base + j * C, C)], wsem[b])

        for b in range(min(NB, n_full)):
            gd(b, b).start()

        def tri(i, carry):
            for b in range(NB):
                j = i * NB + b

                @pl.when(j < n_full)
                def _():
                    gd(j, b).wait()
                    wd(j, b).start()
                    jn = j + NB

                    @pl.when(jn < n_full)
                    def _():
                        wd(j, b).wait()
                        gd(jn, b).start()

            return carry

        lax.fori_loop(0, n_tri, tri, 0)
        for b in range(min(NB, n_full)):
            jl = ((n_full - 1 - b) // NB) * NB + b
            wd(jl, b).wait()
        if rem:
            off = base + n_full * C
            pltpu.async_copy(table_hbm.at[idx_all.at[pl.ds(n_full * C, rem)]],
                             rows[0].at[pl.ds(0, rem)], gsem[0]).wait()
            pltpu.sync_copy(rows[0].at[pl.ds(0, rem)],
                            out_hbm.at[pl.ds(off, rem)])

    return pl.kernel(
        body,
        out_type=jax.ShapeDtypeStruct((E, D), jnp.float32),
        mesh=_mesh(),
        scratch_types=[
            pltpu.VMEM((per_w,), jnp.int32),
        ] + [pltpu.VMEM((C, D), jnp.float32)] * 3
        + [pltpu.SemaphoreType.DMA] * 6,
    )(table, idx)


def _sc_scatter_add(m2_chunks, col, n_nodes):
    E = col.shape[0]
    F = 128
    per_t = E // NS
    C = 128
    n_full = per_t // C
    rem = per_t - n_full * C
    SL = 64
    vp = ((n_nodes + NS * SL - 1) // (NS * SL)) * (NS * SL)
    rows_t = vp // NS
    n_sl = rows_t // SL

    def body(m_all, col_hbm, out_hbm,
             accum, zbuf, mb0, mb1, ib0, ib1, irem,
             is0, is1, ms0, ms1, as0, as1, zsem, wsem):
        c = lax.axis_index("c")
        s = lax.axis_index("s")
        mbuf = (mb0, mb1)
        ibuf = (ib0, ib1)
        isem = (is0, is1)
        msem = (ms0, ms1)
        asem = (as0, as1)

        zv = jnp.zeros((16,), jnp.float32)

        def zrow(i, carry):
            for j in range(F // 16):
                zbuf[i, pl.ds(j * 16, 16)] = zv
            return carry

        lax.fori_loop(0, SL, zrow, 0)

        for k in range(2):
            q = 2 * c + k

            def zd(t):
                return pltpu.make_async_copy(
                    zbuf, accum.at[pl.ds(s * rows_t + t * SL, SL)], zsem)

            def zfire(t, carry):
                zd(t).start()
                return carry

            def zdrain(t, carry):
                zd(t).wait()
                return carry

            lax.fori_loop(0, n_sl, zfire, 0)
            lax.fori_loop(0, n_sl, zdrain, 0)
            plsc.subcore_barrier()

            ebase = s * per_t

            def ld(j, b):
                off = ebase + j * C
                return (pltpu.make_async_copy(
                            col_hbm.at[pl.ds(off, C)], ibuf[b], isem[b]),
                        pltpu.make_async_copy(
                            m_all.at[q, pl.ds(off, C)], mbuf[b], msem[b]))

            def ad(b):
                return pltpu.make_async_copy(
                    mbuf[b], accum.at[ibuf[b]], asem[b])

            for b in range(2):
                di, dm = ld(b, b)
                di.start()
                dm.start()

            def step(i, carry):
                for b in range(2):
                    j = 2 * i + b

                    @pl.when(j < n_full)
                    def _():
                        di, dm = ld(j, b)
                        di.wait()
                        dm.wait()
                        ad(b).start(add=True)
                        jn = j + 2

                        @pl.when(jn < n_full)
                        def _():
                            ad(b).wait()
                            di2, dm2 = ld(jn, b)
                            di2.start()
                            dm2.start()

                return carry

            lax.fori_loop(0, (n_full + 1) // 2, step, 0)
            for b in range(min(2, n_full)):
                ad(b).wait()
            if rem:
                off = ebase + n_full * C
                pltpu.sync_copy(col_hbm.at[pl.ds(off, rem)], irem)
                pltpu.sync_copy(m_all.at[q, pl.ds(off, rem)],
                                mb0.at[pl.ds(0, rem)])
                pltpu.sync_copy(mb0.at[pl.ds(0, rem)], accum.at[irem],
                                add=True)

            plsc.subcore_barrier()

            def wd(t):
                off = s * rows_t + t * SL
                return pltpu.make_async_copy(
                    accum.at[pl.ds(off, SL)], out_hbm.at[q, pl.ds(off, SL)],
                    wsem)

            def wfire(t, carry):
                wd(t).start()
                return carry

            def wdrain(t, carry):
                wd(t).wait()
                return carry

            lax.fori_loop(0, n_sl, wfire, 0)
            lax.fori_loop(0, n_sl, wdrain, 0)
            plsc.subcore_barrier()

    return pl.kernel(
        body,
        out_type=jax.ShapeDtypeStruct((4, vp, F), jnp.float32),
        mesh=_mesh(),
        scratch_types=[
            pltpu.VMEM_SHARED((vp, F), jnp.float32),
            pltpu.VMEM((SL, F), jnp.float32),
            pltpu.VMEM((C, F), jnp.float32),
            pltpu.VMEM((C, F), jnp.float32),
            pltpu.VMEM((C,), jnp.int32),
            pltpu.VMEM((C,), jnp.int32),
            pltpu.VMEM((max(rem, 8),), jnp.int32),
        ] + [pltpu.SemaphoreType.DMA] * 8,
    )(m2_chunks, col)


def _sc_counts(col, n_nodes):
    E = col.shape[0]
    F = 128
    per_t = E // (NC * NS)
    C = 128
    n_full = per_t // C
    rem = per_t - n_full * C
    SL = 128
    vp = ((n_nodes + NS * SL - 1) // (NS * SL)) * (NS * SL)
    rows_t = vp // NS
    n_sl = rows_t // SL

    def body(col_hbm, out_hbm, accum, zbuf, ones_v, ib0, ib1, irem,
             is0, is1, as0, as1, zsem, wsem):
        c = lax.axis_index("c")
        s = lax.axis_index("s")
        ibuf = (ib0, ib1)
        isem = (is0, is1)
        asem = (as0, as1)
        zv = jnp.zeros((16,), jnp.float32)
        ov = jnp.ones((16,), jnp.float32)

        def zrow(i, carry):
            for j in range(F // 16):
                zbuf[i, pl.ds(j * 16, 16)] = zv
            return carry

        lax.fori_loop(0, SL, zrow, 0)

        def orow(i, carry):
            for j in range(F // 16):
                ones_v[i, pl.ds(j * 16, 16)] = ov
            return carry

        lax.fori_loop(0, C, orow, 0)

        def zd(t):
            return pltpu.make_async_copy(
                zbuf, accum.at[pl.ds(s * rows_t + t * SL, SL)], zsem)

        def zfire(t, carry):
            zd(t).start()
            return carry

        def zdrain(t, carry):
            zd(t).wait()
            return carry

        lax.fori_loop(0, n_sl, zfire, 0)
        lax.fori_loop(0, n_sl, zdrain, 0)
        plsc.subcore_barrier()

        ebase = (c * NS + s) * per_t

        def ld(j, b):
            return pltpu.make_async_copy(
                col_hbm.at[pl.ds(ebase + j * C, C)], ibuf[b], isem[b])

        def ad(b):
            return pltpu.make_async_copy(ones_v, accum.at[ibuf[b]], asem[b])

        for b in range(2):
            ld(b, b).start()

        def step(i, carry):
            for b in range(2):
                j = 2 * i + b
                ld(j, b).wait()
                ad(b).start(add=True)
                jn = j + 2

                @pl.when(jn < n_full)
                def _():
                    ad(b).wait()
                    ld(jn, b).start()

            return carry

        lax.fori_loop(0, n_full // 2, step, 0)
        if n_full % 2:
            ld(n_full - 1, 0).wait()
            ad(0).start(add=True)
        for b in range(2):
            ad(b).wait()
        if rem:
            off = ebase + n_full * C
            pltpu.sync_copy(col_hbm.at[pl.ds(off, rem)], irem)
            pltpu.sync_copy(ones_v.at[pl.ds(0, rem)], accum.at[irem],
                            add=True)
        plsc.subcore_barrier()

        def wd(t):
            off = s * rows_t + t * SL
            return pltpu.make_async_copy(
                accum.at[pl.ds(off, SL)], out_hbm.at[c, pl.ds(off, SL)],
                wsem)

        def wfire(t, carry):
            wd(t).start()
            return carry

        def wdrain(t, carry):
            wd(t).wait()
            return carry

        lax.fori_loop(0, n_sl, wfire, 0)
        lax.fori_loop(0, n_sl, wdrain, 0)

    return pl.kernel(
        body,
        out_type=jax.ShapeDtypeStruct((NC, vp, F), jnp.float32),
        mesh=_mesh(),
        scratch_types=[
            pltpu.VMEM_SHARED((vp, F), jnp.float32),
            pltpu.VMEM((SL, F), jnp.float32),
            pltpu.VMEM((C, F), jnp.float32),
            pltpu.VMEM((C,), jnp.int32),
            pltpu.VMEM((C,), jnp.int32),
            pltpu.VMEM((max(rem, 8),), jnp.int32),
        ] + [pltpu.SemaphoreType.DMA] * 6,
    )(col)


BN = 2000
BE = 1600
E_SPLIT = 76800


def _full(shape):
    return pl.BlockSpec(shape, lambda i: (0,) * len(shape))


def _rows(block):
    return pl.BlockSpec(block, lambda i: (i,) + (0,) * (len(block) - 1))


def _node_encoder(mol_x_pad, aW, ab, ag, abeta, npW, npb, npg, npbeta,
                  w1a0):
    n = mol_x_pad.shape[0]
    K = mol_x_pad.shape[1]

    def body(x_ref, aW_r, ab_r, ag_r, abeta_r, npW_r, npb_r, npg_r,
             npbeta_r, w1a0_r, x0_ref, xa0_ref):
        a = _mmT(x_ref[...], aW_r[...]) + ab_r[...]
        a = _gelu(_ln(a, ag_r[...], abeta_r[...]))
        x0 = _mmT(a, npW_r[...]) + npb_r[...]
        x0 = _gelu(_ln(x0, npg_r[...], npbeta_r[...]))
        x0_ref[...] = x0
        xa0_ref[...] = _mmT(x0, w1a0_r[...])

    return pl.pallas_call(
        body,
        grid=(n // BN,),
        in_specs=[
            _rows((BN, K)),
            _full((H // 2, K)), _full((1, H // 2)), _full((1, H // 2)),
            _full((1, H // 2)),
            _full((H, H // 2)), _full((1, H)), _full((1, H)), _full((1, H)),
            _full((H, H)),
        ],
        out_specs=[_rows((BN, H)), _rows((BN, H))],
        out_shape=[jax.ShapeDtypeStruct((n, H), jnp.float32),
                   jax.ShapeDtypeStruct((n, H), jnp.float32)],
    )(mol_x_pad, aW, ab, ag, abeta, npW, npb, npg, npbeta, w1a0)


def _edge_mlp0(attr, scal, g0, bW, bb, bg, bbeta, s4, sb_w, sb, sg, sbeta,
               epW, epb, epg, epbeta, w1b, b1, g1, beta1, w2, b2, g2,
               beta2):
    e = attr.shape[0]

    def body(attr_r, scal_r, g_r, bW_r, bb_r, bg_r, bbeta_r, s4_r, sbw_r,
             sb_r, sg_r, sbeta_r, epW_r, epb_r, epg_r, epbeta_r, w1b_r,
             b1_r, g1_r, beta1_r, w2_r, b2_r, g2_r, beta2_r,
             ea_ref, m_ref):
        bond = _mmT(attr_r[...], bW_r[...]) + bb_r[...]
        bond = _gelu(_ln(bond, bg_r[...], bbeta_r[...]))
        spat = (_mmT(scal_r[...], s4_r[...])
                + _mmT(bond, sbw_r[...]) + sb_r[...])
        spat = _gelu(_ln(spat, sg_r[...], sbeta_r[...]))
        ea = _mmT(spat, epW_r[...]) + epb_r[...]
        ea = _gelu(_ln(ea, epg_r[...], epbeta_r[...]))
        ea_ref[...] = ea
        t = g_r[...] + _mmT(ea, w1b_r[...]) + b1_r[...]
        t = _gelu(_ln(t, g1_r[...], beta1_r[...]))
        m = _mmT(t, w2_r[...]) + b2_r[...]
        m = _gelu(_ln(m, g2_r[...], beta2_r[...]))
        for q in range(4):
            m_ref[q] = m[:, q * 128:(q + 1) * 128]

    return pl.pallas_call(
        body,
        grid=(e // BE,),
        in_specs=[
            _rows((BE, 8)), _rows((BE, 8)), _rows((BE, H)),
            _full((128, 8)), _full((1, 128)), _full((1, 128)),
            _full((1, 128)),
            _full((256, 8)), _full((256, 128)), _full((1, 256)),
            _full((1, 256)), _full((1, 256)),
            _full((H, 256)), _full((1, H)), _full((1, H)), _full((1, H)),
            _full((H, H)), _full((1, H)), _full((1, H)), _full((1, H)),
            _full((H, H)), _full((1, H)), _full((1, H)), _full((1, H)),
        ],
        out_specs=[_rows((BE, H)),
                   pl.BlockSpec((4, BE, 128), lambda i: (0, i, 0))],
        out_shape=[jax.ShapeDtypeStruct((e, H), jnp.float32),
                   jax.ShapeDtypeStruct((4, e, 128), jnp.float32)],
    )(attr, scal, g0, bW, bb, bg, bbeta, s4, sb_w, sb, sg, sbeta, epW,
      epb, epg, epbeta, w1b, b1, g1, beta1, w2, b2, g2, beta2)


def _edge_mlp(ea, g, w1b, b1, g1, beta1, w2, b2, g2, beta2):
    e = ea.shape[0]

    def body(ea_r, g_r, w1b_r, b1_r, g1_r, beta1_r, w2_r, b2_r, g2_r,
             beta2_r, m_ref):
        t = g_r[...] + _mmT(ea_r[...], w1b_r[...]) + b1_r[...]
        t = _gelu(_ln(t, g1_r[...], beta1_r[...]))
        m = _mmT(t, w2_r[...]) + b2_r[...]
        m = _gelu(_ln(m, g2_r[...], beta2_r[...]))
        for q in range(4):
            m_ref[q] = m[:, q * 128:(q + 1) * 128]

    return pl.pallas_call(
        body,
        grid=(e // BE,),
        in_specs=[
            _rows((BE, H)), _rows((BE, H)),
            _full((H, H)), _full((1, H)), _full((1, H)), _full((1, H)),
            _full((H, H)), _full((1, H)), _full((1, H)), _full((1, H)),
        ],
        out_specs=pl.BlockSpec((4, BE, 128), lambda i: (0, i, 0)),
        out_shape=jax.ShapeDtypeStruct((4, e, 128), jnp.float32),
    )(ea, g, w1b, b1, g1, beta1, w2, b2, g2, beta2)


def _x_update(x, s_a, s_b, cnt, w1a):
    n = x.shape[0]

    def body(x_r, sa_r, sb_r, c_r, w1a_r, xn_ref, xa_ref):
        sva = sa_r[...]
        svb = sb_r[...]
        s = jnp.concatenate([sva[q] + svb[q] for q in range(4)], axis=-1)
        cv = c_r[...]
        cnt = cv[0][:, 0:1] + cv[1][:, 0:1]
        xn = x_r[...] + s / jnp.maximum(cnt, 1.0)
        xn_ref[...] = xn
        xa_ref[...] = _mmT(xn, w1a_r[...])

    return pl.pallas_call(
        body,
        grid=(n // BN,),
        in_specs=[_rows((BN, H)),
                  pl.BlockSpec((4, BN, 128), lambda i: (0, i, 0)),
                  pl.BlockSpec((4, BN, 128), lambda i: (0, i, 0)),
                  pl.BlockSpec((2, BN, 128), lambda i: (0, i, 0)),
                  _full((H, H))],
        out_specs=[_rows((BN, H)), _rows((BN, H))],
        out_shape=[jax.ShapeDtypeStruct((n, H), jnp.float32),
                   jax.ShapeDtypeStruct((n, H), jnp.float32)],
    )(x, s_a, s_b, cnt, w1a)


def _seq_branch(emb, seqW, seqb, wv0, bv0, wo0, bo0, g10, beta10, w10,
                b10, w20, b20, g20, beta20, wv1, bv1, wo1, bo1, g11,
                beta11, w11, b11, w21, b21, g21, beta21, fusWs, fusb):
    def body(emb_r, seqW_r, seqb_r, wv0_r, bv0_r, wo0_r, bo0_r, g10_r,
             beta10_r, w10_r, b10_r, w20_r, b20_r, g20_r, beta20_r,
             wv1_r, bv1_r, wo1_r, bo1_r, g11_r, beta11_r, w11_r, b11_r,
             w21_r, b21_r, g21_r, beta21_r, fusWs_r, fusb_r, c_ref):
        h = _mmT(emb_r[...], seqW_r[...]) + seqb_r[...]
        for (wv, bv, wo, bo, G1, B1, w1, b1, w2, b2, G2, B2) in (
                (wv0_r, bv0_r, wo0_r, bo0_r, g10_r, beta10_r, w10_r,
                 b10_r, w20_r, b20_r, g20_r, beta20_r),
                (wv1_r, bv1_r, wo1_r, bo1_r, g11_r, beta11_r, w11_r,
                 b11_r, w21_r, b21_r, g21_r, beta21_r)):
            v = _mmT(h, wv[...]) + bv[...]
            a = _mmT(v, wo[...]) + bo[...]
            h = _ln(h + a, G1[...], B1[...])
            ff = jnp.maximum(_mmT(h, w1[...]) + b1[...], 0.0)
            ff = _mmT(ff, w2[...]) + b2[...]
            h = _ln(h + ff, G2[...], B2[...])
        sf = jnp.mean(h, axis=0, keepdims=True)
        c_ref[...] = _mmT(sf, fusWs_r[...]) + fusb_r[...]

    return pl.pallas_call(
        body,
        out_shape=jax.ShapeDtypeStruct((1, H), jnp.float32),
    )(emb, seqW, seqb, wv0, bv0, wo0, bo0, g10, beta10, w10, b10, w20,
      b20, g20, beta20, wv1, bv1, wo1, bo1, g11, beta11, w11, b11, w21,
      b21, g21, beta21, fusWs, fusb)


def _final(x, s_a, s_b, cnt, c_row, gng, gnbeta, fusWm,
           resW, resb, resg, resbeta):
    n = x.shape[0]

    def body(x_r, sa_r, sb_r, cn_r, c_r, gng_r, gnbeta_r,
             fusWm_r, resW_r, resb_r, resg_r, resbeta_r, out_ref):
        sva = sa_r[...]
        svb = sb_r[...]
        s = jnp.concatenate([sva[q] + svb[q] for q in range(4)], axis=-1)
        cv = cn_r[...]
        cnt = cv[0][:, 0:1] + cv[1][:, 0:1]
        x3 = x_r[...] + s / jnp.maximum(cnt, 1.0)
        xg = _ln(x3, gng_r[...], gnbeta_r[...])
        y = _mmT(xg, fusWm_r[...]) + c_r[...]
        z = _mmT(y, resW_r[...]) + resb_r[...]
        z = jnp.where(z != z, 0.0,
                      jnp.where(z == jnp.inf, 1e5,
                                jnp.where(z == -jnp.inf, -1e5, z)))
        z = _ln(z, resg_r[...], resbeta_r[...])
        z = jnp.clip(z, -10.0, 10.0)
        z = _gelu(z)
        out_ref[...] = y + z

    return pl.pallas_call(
        body,
        grid=(n // BN,),
        in_specs=[_rows((BN, H)),
                  pl.BlockSpec((4, BN, 128), lambda i: (0, i, 0)),
                  pl.BlockSpec((4, BN, 128), lambda i: (0, i, 0)),
                  pl.BlockSpec((2, BN, 128), lambda i: (0, i, 0)),
                  _full((1, H)), _full((1, H)),
                  _full((1, H)), _full((H, H)), _full((H, H)), _full((1, H)),
                  _full((1, H)), _full((1, H))],
        out_specs=_rows((BN, H)),
        out_shape=jax.ShapeDtypeStruct((n, H), jnp.float32),
    )(x, s_a, s_b, cnt, c_row, gng, gnbeta, fusWm, resW,
      resb, resg, resbeta)


def kernel(mol_x, mol_edge_attr, mol_dist, mol_theta, mol_phi, mol_tau,
           mol_embedding, mol_edge_index, params):
    p = params
    n = mol_x.shape[0]
    row = mol_edge_index[0].astype(jnp.int32)
    col = mol_edge_index[1].astype(jnp.int32)

    def r2(v):
        return v.reshape(1, -1)

    mol_x_pad = jnp.pad(mol_x, ((0, 0), (0, 80 - mol_x.shape[1])))
    aW = jnp.pad(p['atom_W'], ((0, 0), (0, 80 - p['atom_W'].shape[1])))
    scal = jnp.stack([mol_dist, mol_theta, mol_phi, mol_tau], axis=1)
    scal = jnp.pad(scal, ((0, 0), (0, 4)))
    s4 = jnp.pad(p['spat_W'][:, :4], ((0, 0), (0, 4)))
    sb_w = p['spat_W'][:, 4:]

    w1a = [p[f'l{i}_W1'][:, :H] for i in range(3)]
    w1b = [p[f'l{i}_W1'][:, H:] for i in range(3)]
    w2 = [p[f'l{i}_W2'] for i in range(3)]

    x0, xa = _node_encoder(
        mol_x_pad, aW, r2(p['atom_b']), r2(p['atom_g']), r2(p['atom_beta']),
        p['np_W'], r2(p['np_b']), r2(p['np_g']), r2(p['np_beta']),
        w1a[0])

    cnt = _sc_counts(col, n)

    emb = mol_embedding.reshape(mol_embedding.shape[1], mol_embedding.shape[2])
    tw = []
    for i in range(2):
        win, bin_ = p[f't{i}_Win'], p[f't{i}_bin']
        tw += [win[2 * H:3 * H], r2(bin_[2 * H:3 * H]),
               p[f't{i}_Wout'], r2(p[f't{i}_bout']),
               r2(p[f't{i}_g1']), r2(p[f't{i}_beta1']),
               p[f't{i}_W1'], r2(p[f't{i}_b1']),
               p[f't{i}_W2'], r2(p[f't{i}_b2']),
               r2(p[f't{i}_g2']), r2(p[f't{i}_beta2'])]
    c_row = _seq_branch(emb, p['seq_W'], r2(p['seq_b']), *tw,
                        p['fus_W'][:, H:], r2(p['fus_b']))

    SP = E_SPLIT
    row_h = (row[:SP], row[SP:])
    col_h = (col[:SP], col[SP:])
    attr_h = (mol_edge_attr[:SP], mol_edge_attr[SP:])
    scal_h = (scal[:SP], scal[SP:])

    ea_h = [None, None]
    x = x0
    for i in range(3):
        m_h = [None, None]
        for hh in range(2):
            g = _sc_gather(xa, row_h[hh])
            if i == 0:
                ea_h[hh], m_h[hh] = _edge_mlp0(
                    attr_h[hh], scal_h[hh], g,
                    p['bond_W'], r2(p['bond_b']), r2(p['bond_g']),
                    r2(p['bond_beta']),
                    s4, sb_w, r2(p['spat_b']), r2(p['spat_g']),
                    r2(p['spat_beta']),
                    p['ep_W'], r2(p['ep_b']), r2(p['ep_g']), r2(p['ep_beta']),
                    w1b[0], r2(p['l0_b1']), r2(p['l0_g1']), r2(p['l0_beta1']),
                    w2[0], r2(p['l0_b2']), r2(p['l0_g2']), r2(p['l0_beta2']))
            else:
                m_h[hh] = _edge_mlp(
                    ea_h[hh], g, w1b[i], r2(p[f'l{i}_b1']), r2(p[f'l{i}_g1']),
                    r2(p[f'l{i}_beta1']), w2[i], r2(p[f'l{i}_b2']),
                    r2(p[f'l{i}_g2']), r2(p[f'l{i}_beta2']))
        s_a = _sc_scatter_add(m_h[0], col_h[0], n)
        s_b = _sc_scatter_add(m_h[1], col_h[1], n)
        if i < 2:
            x, xa = _x_update(x, s_a, s_b, cnt, w1a[i + 1])

    out = _final(x, s_a, s_b, cnt, c_row,
                 r2(p['gn_g']), r2(p['gn_beta']), p['fus_W'][:, :H],
                 p['res_W'], r2(p['res_b']), r2(p['res_g']),
                 r2(p['res_beta']))
    return out
